# Initial kernel scaffold; baseline (speedup 1.0000x reference)
import jax
import jax.numpy as jnp
from jax.experimental import pallas as pl


def kernel(at_no, pos, edge_index, edge_index_full, params):
    raise NotImplementedError("write your pallas kernel here")


# speedup vs baseline: 2.1133x; 2.1133x over previous
import functools

import jax
import jax.numpy as jnp
from jax import lax
from jax.experimental import pallas as pl
from jax.experimental.pallas import tpu as pltpu
from jax.experimental.pallas import tpu_sc as plsc

D = 128
NB = 20
CUT = 5.0
SPH = 9
SV = 16
W = 160
RW = 32
HID = 64
BLK = 32
N = 10000
NP = 10240
E = 160000
EP = 163840
CH = 64
NTILES = 32
EPT = EP // NTILES
NIT = EPT // CH
RPS = NP // 16
NR = 10112
RPS_S = NR // 16
BN = 1024
BE = 4096

@functools.cache
def _mesh():
    return plsc.VectorSubcoreMesh(core_axis_name="c", subcore_axis_name="s")


def _sig(x):
    return 1.0 / (1.0 + jnp.exp(-x))


def _silu(x):
    return x * _sig(x)


_SCHUNKS = [(k * CH, CH) for k in range(RPS_S // CH)] + [
    ((RPS_S // CH) * CH, RPS_S - (RPS_S // CH) * CH)]


def _sc_dvec_body(pos_h, se_h, de_h, sf_h, df_h, dve_h, dvf_h,
                  spos, ia, ib, pa, dv):
    s = lax.axis_index("s")
    wid = s * 2 + lax.axis_index("c")
    for off, ln in _SCHUNKS:
        pltpu.sync_copy(pos_h.at[pl.ds(s * RPS_S + off, ln)],
                        pa.at[pl.ds(0, ln)])
        pltpu.sync_copy(pa.at[pl.ds(0, ln)],
                        spos.at[pl.ds(s * RPS_S + off, ln)])
    plsc.subcore_barrier()

    def do_list(src_h, dst_h, out_h):
        def it_body(it, _):
            base = wid * EPT + it * CH
            pltpu.sync_copy(src_h.at[pl.ds(base, CH)], ia)
            pltpu.sync_copy(dst_h.at[pl.ds(base, CH)], ib)
            pltpu.sync_copy(spos.at[ia], pa)

            def row1(r, c):
                dv[r, :] = -pa[r, :]
                return c
            lax.fori_loop(0, CH, row1, 0)
            pltpu.sync_copy(spos.at[ib], pa)

            def row2(r, c):
                dv[r, :] = dv[r, :] + pa[r, :]
                return c
            lax.fori_loop(0, CH, row2, 0)
            pltpu.sync_copy(dv, out_h.at[pl.ds(base, CH)])
            return _
        lax.fori_loop(0, NIT, it_body, 0)

    do_list(se_h, de_h, dve_h)
    do_list(sf_h, df_h, dvf_h)


def _sc_dvec(pos_p, se, de, sf, df):
    fn = pl.kernel(
        _sc_dvec_body,
        out_type=[jax.ShapeDtypeStruct((EP, SV), jnp.float32),
                  jax.ShapeDtypeStruct((EP, SV), jnp.float32)],
        mesh=_mesh(),
        scratch_types=[pltpu.VMEM_SHARED((NR, SV), jnp.float32),
                       pltpu.VMEM((CH,), jnp.int32),
                       pltpu.VMEM((CH,), jnp.int32),
                       pltpu.VMEM((CH, SV), jnp.float32),
                       pltpu.VMEM((CH, SV), jnp.float32)],
    )
    return fn(pos_p, se, de, sf, df)


def _sc_msg_body(inv_h, flts_h, src_h, dst_h, sagg_h,
                 accs, iv_s, iv_d, ginv, gflt, ms):
    c = lax.axis_index("c")
    s = lax.axis_index("s")
    wid = s * 2 + c

    def zrow(r, carry):
        for j in range(8):
            ms[r, pl.ds(16 * j, 16)] = jnp.zeros((16,), jnp.float32)
        return carry
    lax.fori_loop(0, CH, zrow, 0)
    for off, ln in _SCHUNKS:
        pltpu.sync_copy(ms.at[pl.ds(0, ln)],
                        accs.at[pl.ds(s * RPS_S + off, ln)])
    plsc.subcore_barrier()

    def it_body(it, _):
        base = wid * EPT + it * CH
        pltpu.sync_copy(src_h.at[pl.ds(base, CH)], iv_s)
        pltpu.sync_copy(dst_h.at[pl.ds(base, CH)], iv_d)
        pltpu.sync_copy(inv_h.at[iv_s], ginv)
        pltpu.sync_copy(flts_h.at[pl.ds(base, CH)], gflt)

        def row(r, carry):
            for j in range(8):
                ms[r, pl.ds(16 * j, 16)] = (ginv[r, pl.ds(16 * j, 16)]
                                            * gflt[r, pl.ds(16 * j, 16)])
            return carry
        lax.fori_loop(0, CH, row, 0)
        pltpu.sync_copy(ms, accs.at[iv_d], add=True)
        return _
    lax.fori_loop(0, NIT, it_body, 0)
    plsc.subcore_barrier()
    for off, ln in _SCHUNKS:
        pltpu.sync_copy(accs.at[pl.ds(s * RPS_S + off, ln)],
                        ms.at[pl.ds(0, ln)])
        pltpu.sync_copy(ms.at[pl.ds(0, ln)],
                        sagg_h.at[c, pl.ds(s * RPS_S + off, ln)])


@functools.cache
def _sc_msg_fn():
    return pl.kernel(
        _sc_msg_body,
        out_type=[jax.ShapeDtypeStruct((2, NP, D), jnp.float32)],
        mesh=_mesh(),
        scratch_types=[pltpu.VMEM_SHARED((NR, D), jnp.float32),
                       pltpu.VMEM((CH,), jnp.int32),
                       pltpu.VMEM((CH,), jnp.int32),
                       pltpu.VMEM((CH, D), jnp.float32),
                       pltpu.VMEM((CH, D), jnp.float32),
                       pltpu.VMEM((CH, D), jnp.float32)],
    )


def _sc_msg(invs, flts, src, dst):
    return _sc_msg_fn()(invs, flts, src, dst)[0]


def _sc_pair_body(tab_h, gc_h, gi_h, si_h, agg_h,
                  acc, iv_g, iv_s, gt, gcv, ef):
    c = lax.axis_index("c")
    s = lax.axis_index("s")
    wid = s * 2 + c

    def zrow(r, carry):
        ef[r, :] = jnp.zeros((16,), jnp.float32)
        return carry
    lax.fori_loop(0, CH, zrow, 0)
    for off, ln in _SCHUNKS:
        pltpu.sync_copy(ef.at[pl.ds(0, ln)],
                        acc.at[pl.ds(s * RPS_S + off, ln)])
    plsc.subcore_barrier()

    def it_body(it, _):
        base = wid * EPT + it * CH
        pltpu.sync_copy(gi_h.at[pl.ds(base, CH)], iv_g)
        pltpu.sync_copy(si_h.at[pl.ds(base, CH)], iv_s)
        pltpu.sync_copy(tab_h.at[iv_g], gt)
        pltpu.sync_copy(gc_h.at[pl.ds(base, CH)], gcv)

        def row(r, carry):
            ef[r, :] = (gcv[r, pl.ds(0, 16)] * gt[r, pl.ds(0, 16)]
                        + gcv[r, pl.ds(16, 16)] * gt[r, pl.ds(16, 16)])
            return carry
        lax.fori_loop(0, CH, row, 0)
        pltpu.sync_copy(ef, acc.at[iv_s], add=True)
        return _
    lax.fori_loop(0, NIT, it_body, 0)
    plsc.subcore_barrier()
    for off, ln in _SCHUNKS:
        pltpu.sync_copy(acc.at[pl.ds(s * RPS_S + off, ln)],
                        ef.at[pl.ds(0, ln)])
        pltpu.sync_copy(ef.at[pl.ds(0, ln)],
                        agg_h.at[c, pl.ds(s * RPS_S + off, ln)])


@functools.cache
def _sc_pair_fn():
    return pl.kernel(
        _sc_pair_body,
        out_type=[jax.ShapeDtypeStruct((2, NP, SV), jnp.float32)],
        mesh=_mesh(),
        scratch_types=[pltpu.VMEM_SHARED((NR, SV), jnp.float32),
                       pltpu.VMEM((CH,), jnp.int32),
                       pltpu.VMEM((CH,), jnp.int32),
                       pltpu.VMEM((CH, D), jnp.float32),
                       pltpu.VMEM((CH, RW), jnp.float32),
                       pltpu.VMEM((CH, SV), jnp.float32)],
    )


def _sc_pair(tab, gc, gidx, sidx):
    return _sc_pair_fn()(tab, gc, gidx, sidx)[0]


def _tc(body, grid, in_specs, out_specs, out_shapes):
    single = not isinstance(out_shapes, (list, tuple))
    if single:
        out_shapes = [out_shapes]
    fn = pl.pallas_call(
        body, grid=grid, in_specs=list(in_specs),
        out_specs=list(out_specs), out_shape=list(out_shapes))
    if single:
        return lambda *a: fn(*a)[0]
    return fn


def _full(shape):
    return pl.BlockSpec(shape, lambda i: (0,) * len(shape))


def _rows(bs, width):
    return pl.BlockSpec((bs, width), lambda i: (i, 0))


def _embed_inv_body(an_ref, emb_ref, w1_ref, w2_ref, bv_ref,
                    xs_ref, invs_ref, vtab_ref):
    an = an_ref[...]
    lanes = lax.broadcasted_iota(jnp.int32, (BN, D), 1)
    oh = (an == lanes).astype(jnp.float32)
    xs = jnp.dot(oh, emb_ref[...], preferred_element_type=jnp.float32)
    xs_ref[...] = xs
    b1 = bv_ref[0:1, 0:D]
    b2 = bv_ref[1:2, :]
    t = _silu(jnp.dot(xs, w1_ref[...], preferred_element_type=jnp.float32) + b1)
    inv = jnp.dot(t, w2_ref[...], preferred_element_type=jnp.float32) + b2
    invs_ref[...] = inv[:, 0:D]
    vtab_ref[...] = jnp.concatenate(
        [jnp.zeros((BN, SV), jnp.float32), inv[:, 144:160],
         jnp.zeros((BN, 96), jnp.float32)], axis=1)


def _tc_embed_inv(an_p, emb_p, w1, w2p, bv):
    return _tc(
        _embed_inv_body, (NP // BN,),
        [_rows(BN, 1), _full((D, D)), _full((D, D)), _full((D, W)),
         _full((8, W))],
        [_rows(BN, D), _rows(BN, D), _rows(BN, D)],
        (jax.ShapeDtypeStruct((NP, D), jnp.float32),
         jax.ShapeDtypeStruct((NP, D), jnp.float32),
         jax.ShapeDtypeStruct((NP, D), jnp.float32)),
    )(an_p, emb_p, w1, w2p, bv)


def _geom_common(dv):
    x = dv[:, 0:1]
    y = dv[:, 1:2]
    z = dv[:, 2:3]
    d = jnp.sqrt(x * x + y * y + z * z + 1e-12)
    nlane = (lax.broadcasted_iota(jnp.int32, (dv.shape[0], RW), 1)
             .astype(jnp.float32) + 1.0)
    rbf = jnp.sin(nlane * (jnp.pi / CUT) * d) / d
    dmin = jnp.minimum(d, CUT)
    fcut = 0.5 * (jnp.cos(jnp.pi * dmin / CUT) + 1.0) * (d < CUT)
    return x / d, y / d, z / d, rbf, fcut


def _edge_geom_body(dv_ref, rbfcf_ref, rsh_ref):
    pid = pl.program_id(0)
    dv = dv_ref[...]
    ux, uy, uz, rbf, fcut = _geom_common(dv)
    gid = pid * BE + lax.broadcasted_iota(jnp.int32, (BE, 1), 0)
    mask = (gid < E).astype(jnp.float32)
    lanes = lax.broadcasted_iota(jnp.int32, (BE, RW), 1)
    fc = fcut * mask
    rbfcf = jnp.where(lanes < NB, rbf * fc, jnp.where(lanes == NB, fc, 0.0))
    rbfcf_ref[...] = rbfcf
    l16 = lax.broadcasted_iota(jnp.int32, (BE, SV), 1)
    r = jnp.where(l16 == 0, 1.0, 0.0)
    for i, t in enumerate([ux, uy, uz, ux * uy, uy * uz,
                           3.0 * uz * uz - 1.0, ux * uz, ux * ux - uy * uy]):
        r = jnp.where(l16 == i + 1, t, r)
    rsh_ref[...] = r


def _tc_edge_geom(dve):
    return _tc(
        _edge_geom_body, (EP // BE,),
        [_rows(BE, SV)],
        [_rows(BE, RW), _rows(BE, SV)],
        (jax.ShapeDtypeStruct((EP, RW), jnp.float32),
         jax.ShapeDtypeStruct((EP, SV), jnp.float32)),
    )(dve)


def _full_geom_body(dv_ref, wc_ref, gc_ref):
    pid = pl.program_id(0)
    dv = dv_ref[...]
    _, _, _, rbf, fcut = _geom_common(dv)
    gid = pid * BE + lax.broadcasted_iota(jnp.int32, (BE, 1), 0)
    mask = (gid < E).astype(jnp.float32)
    lanes = lax.broadcasted_iota(jnp.int32, (BE, RW), 1)
    frbf = jnp.where(lanes < NB, rbf * fcut * mask, 0.0)
    gc_ref[...] = jnp.dot(frbf, wc_ref[...],
                          preferred_element_type=jnp.float32)


def _tc_full_geom(dvf, wcomb):
    return _tc(
        _full_geom_body, (EP // BE,),
        [_rows(BE, SV), _full((RW, RW))],
        [_rows(BE, RW)],
        jax.ShapeDtypeStruct((EP, RW), jnp.float32),
    )(dvf, wcomb)


def _filt_body(rb_ref, rsh_ref, wr_ref, flts_ref, etab_ref):
    flt = jnp.dot(rb_ref[...], wr_ref[...], preferred_element_type=jnp.float32)
    flts_ref[...] = flt[:, 0:D]
    etab_ref[...] = jnp.concatenate(
        [flt[:, D:D + SV], flt[:, D + SV:W] * rsh_ref[...]], axis=1)


def _tc_filt(rbfcf, rsh, wrp):
    return _tc(
        _filt_body, (EP // BE,),
        [_rows(BE, RW), _rows(BE, SV), _full((RW, W))],
        [_rows(BE, D), _rows(BE, RW)],
        (jax.ShapeDtypeStruct((EP, D), jnp.float32),
         jax.ShapeDtypeStruct((EP, RW), jnp.float32)),
    )(rbfcf, rsh, wrp)


def _make_update_body(has_sph, emit_inv):
    def body(*refs):
        i = 0
        xs_ref = refs[i]; i += 1
        xv_ref = refs[i]; i += 1
        sagg_ref = refs[i]; i += 1
        vagg_ref = refs[i]; i += 1
        if has_sph == 2:
            nin_ref = refs[i]; i += 1
            xva_ref = refs[i]; i += 1
        up_ref = refs[i]; i += 1
        vp_ref = refs[i]; i += 1
        wu1_ref = refs[i]; i += 1
        wu2_ref = refs[i]; i += 1
        msc_ref = refs[i]; i += 1
        if emit_inv:
            w1n_ref = refs[i]; i += 1
            w2n_ref = refs[i]; i += 1
        xs_o = refs[i]; i += 1
        xv_o = refs[i]; i += 1
        if has_sph:
            nsph_o = refs[i]; i += 1
        if emit_inv:
            invs_o = refs[i]; i += 1
            vtab_o = refs[i]; i += 1
        if has_sph == 2:
            xvt_o = refs[i]; i += 1

        f32 = jnp.float32
        xs1 = xs_ref[...] + sagg_ref[0] + sagg_ref[1]
        mu = jnp.mean(xs1, axis=-1, keepdims=True)
        var = jnp.mean((xs1 - mu) ** 2, axis=-1, keepdims=True)
        xs1 = (xs1 - mu) / jnp.sqrt(var + 1e-5)
        xv1 = xv_ref[...] + vagg_ref[0] + vagg_ref[1]
        uv = jnp.dot(xv1, up_ref[...], preferred_element_type=f32)
        vv = jnp.dot(xv1, vp_ref[...], preferred_element_type=f32)
        vnorm = jnp.sqrt(jnp.sum(vv * vv, axis=-1, keepdims=True) + 1e-12)
        wu1b = msc_ref[0:1, 0:D]
        bu1 = msc_ref[1:2, 0:D]
        bu2 = msc_ref[2:3, :]
        t = _silu(jnp.dot(xs1, wu1_ref[...], preferred_element_type=f32)
                  + vnorm * wu1b + bu1)
        a = jnp.dot(t, wu2_ref[...], preferred_element_type=f32) + bu2
        dots = jnp.sum(uv * vv, axis=-1, keepdims=True)
        xs2 = xs1 + a[:, 0:D] + a[:, D:D + 1] * dots
        xv2 = xv1 + a[:, 144:160] * uv
        xs_o[...] = xs2
        xv_o[...] = xv2
        if has_sph:
            wn = msc_ref[3:4, 0:SV]
            ns = xv2 * wn
            if has_sph == 2:
                ns = ns + nin_ref[...]
            nsph_o[...] = ns
        if emit_inv:
            b1n = msc_ref[4:5, 0:D]
            b2n = msc_ref[5:6, :]
            tn = _silu(jnp.dot(xs2, w1n_ref[...], preferred_element_type=f32)
                       + b1n)
            invf = jnp.dot(tn, w2n_ref[...], preferred_element_type=f32) + b2n
            invs_o[...] = invf[:, 0:D]
            vtab_o[...] = jnp.concatenate(
                [invf[:, D:D + SV] * xv2, invf[:, D + SV:W],
                 jnp.zeros((BN, 96), f32)], axis=1)
        if has_sph == 2:
            xvt_o[...] = jnp.concatenate(
                [xva_ref[...], xv2, jnp.zeros((BN, 96), f32)], axis=1)
    return body


def _tc_update(has_sph, emit_inv, xs, xv, sagg, vagg, nin, up, vp, wu1, wu2,
               msc, w1n, w2n, xva=None):
    in_arrays = [xs, xv, sagg, vagg]
    in_specs = [_rows(BN, D), _rows(BN, SV),
                pl.BlockSpec((2, BN, D), lambda i: (0, i, 0)),
                pl.BlockSpec((2, BN, SV), lambda i: (0, i, 0))]
    if has_sph == 2:
        in_arrays += [nin, xva]
        in_specs += [_rows(BN, SV), _rows(BN, SV)]
    in_arrays += [up, vp, wu1, wu2, msc]
    in_specs += [_full((SV, SV)), _full((SV, SV)), _full((D, D)),
                 _full((D, W)), _full((8, W))]
    if emit_inv:
        in_arrays += [w1n, w2n]
        in_specs += [_full((D, D)), _full((D, W))]
    out_specs = [_rows(BN, D), _rows(BN, SV)]
    out_shapes = [jax.ShapeDtypeStruct((NP, D), jnp.float32),
                  jax.ShapeDtypeStruct((NP, SV), jnp.float32)]
    if has_sph:
        out_specs.append(_rows(BN, SV))
        out_shapes.append(jax.ShapeDtypeStruct((NP, SV), jnp.float32))
    if emit_inv:
        out_specs += [_rows(BN, D), _rows(BN, D)]
        out_shapes += [jax.ShapeDtypeStruct((NP, D), jnp.float32),
                       jax.ShapeDtypeStruct((NP, D), jnp.float32)]
    if has_sph == 2:
        out_specs.append(_rows(BN, D))
        out_shapes.append(jax.ShapeDtypeStruct((NP, D), jnp.float32))
    return _tc(_make_update_body(has_sph, emit_inv), (NP // BN,),
               in_specs, out_specs, tuple(out_shapes))(*in_arrays)


def _final_body(xs_ref, ns_ref, ea_ref, eb_ref, w1a_ref, w1b_ref, w2_ref,
                wep_ref, msc_ref, out_ref):
    f32 = jnp.float32
    b1 = msc_ref[0:1, :]
    b2 = msc_ref[1:2, 0:BLK]
    e9 = ea_ref[0] + ea_ref[1] + eb_ref[0] + eb_ref[1]
    nh = _silu(jnp.dot(xs_ref[...], w1a_ref[...], preferred_element_type=f32)
               + jnp.dot(ns_ref[...], w1b_ref[...], preferred_element_type=f32)
               + b1)
    out_ref[...] = (jnp.dot(nh, w2_ref[...], preferred_element_type=f32) + b2
                    + jnp.dot(e9, wep_ref[...], preferred_element_type=f32))


def _tc_final(xs, nsph, eagg, ebgg, w1a, w1b, w2, wep, msc):
    return _tc(
        _final_body, (NP // BN,),
        [_rows(BN, D), _rows(BN, SV),
         pl.BlockSpec((2, BN, SV), lambda i: (0, i, 0)),
         pl.BlockSpec((2, BN, SV), lambda i: (0, i, 0)),
         _full((D, HID)), _full((SV, HID)), _full((HID, BLK)),
         _full((SV, BLK)), _full((8, HID))],
        [_rows(BN, BLK)],
        jax.ShapeDtypeStruct((NP, BLK), jnp.float32),
    )(xs, nsph, eagg, ebgg, w1a, w1b, w2, wep, msc)


def _pad_cols(w, b):
    fi = w.shape[0]
    wp = jnp.zeros((fi, W), jnp.float32)
    wp = wp.at[:, 0:D].set(w[:, 0:D])
    wp = wp.at[:, D:D + SPH].set(w[:, D:D + SPH])
    wp = wp.at[:, 144:144 + SPH].set(w[:, D + SPH:D + 2 * SPH])
    bp = jnp.zeros((W,), jnp.float32)
    bp = bp.at[0:D].set(b[0:D])
    bp = bp.at[D:D + SPH].set(b[D:D + SPH])
    bp = bp.at[144:144 + SPH].set(b[D + SPH:D + 2 * SPH])
    return wp, bp


def kernel(at_no, pos, edge_index, edge_index_full, params):
    f32 = jnp.float32
    i32 = jnp.int32

    an_p = jnp.zeros((NP, 1), i32).at[:N, 0].set(at_no.astype(i32))
    pos_p = jnp.zeros((NR, SV), f32).at[:N, :3].set(pos)
    npad = EP - E
    spread = ((jnp.arange(npad, dtype=i32) * 37) % (N - 1)).astype(i32)
    def padidx(a):
        return jnp.concatenate([a.astype(i32), spread])
    se, de = padidx(edge_index[0]), padidx(edge_index[1])
    sf, df = padidx(edge_index_full[0]), padidx(edge_index_full[1])

    emb_p = jnp.zeros((D, D), f32).at[:100].set(params['embed'])
    msg, upd = [], []
    for i in range(3):
        pm = params['msg%d' % i]
        w2p, b2p = _pad_cols(pm['W2'], pm['b2'])
        wrp_, brp = _pad_cols(pm['Wr'], pm['br'])
        wrp = jnp.zeros((RW, W), f32).at[0:NB].set(wrp_).at[NB].set(brp)
        msg.append({'W1': pm['W1'], 'b1': pm['b1'], 'W2p': w2p, 'b2p': b2p,
                    'Wrp': wrp})
        pu = params['upd%d' % i]
        up = jnp.zeros((SV, SV), f32).at[:SPH, :SPH].set(pu['U'])
        vp = jnp.zeros((SV, SV), f32).at[:SPH, :SPH].set(pu['V'])
        wu2 = pu['Wu2']
        wu2p = jnp.zeros((D, W), f32)
        wu2p = wu2p.at[:, 0:D].set(wu2[:, 0:D])
        wu2p = wu2p.at[:, D:D + 1].set(wu2[:, D:D + 1])
        wu2p = wu2p.at[:, 144:144 + SPH].set(wu2[:, D + 1:D + 1 + SPH])
        bu2p = jnp.zeros((W,), f32)
        bu2p = bu2p.at[0:D].set(pu['bu2'][0:D])
        bu2p = bu2p.at[D:D + 1].set(pu['bu2'][D:D + 1])
        bu2p = bu2p.at[144:144 + SPH].set(pu['bu2'][D + 1:D + 1 + SPH])
        upd.append({'U': up, 'V': vp, 'Wu1a': pu['Wu1'][:D, :],
                    'wu1b': pu['Wu1'][D, :], 'bu1': pu['bu1'], 'bu2p': bu2p,
                    'Wu2p': wu2p})
    wn = [params['mat%d' % j]['wn'] for j in range(2)]
    wcomb = jnp.zeros((RW, RW), f32)
    wcomb = wcomb.at[0:NB, 0:SPH].set(params['mat0']['We'])
    wcomb = wcomb.at[0:NB, SV:SV + SPH].set(params['mat1']['We'])
    po = params['out']
    w1a = po['W1'][:D, :]
    w1b = jnp.zeros((SV, HID), f32).at[:SPH].set(po['W1'][D:D + SPH, :])
    wep = jnp.zeros((SV, BLK), f32).at[:SPH].set(po['We'])
    mscf = jnp.zeros((8, HID), f32).at[0].set(po['b1'])
    mscf = mscf.at[1, 0:BLK].set(po['b2'])

    def mk_misc(i):
        m = jnp.zeros((8, W), f32)
        m = m.at[0, 0:D].set(upd[i]['wu1b'])
        m = m.at[1, 0:D].set(upd[i]['bu1'])
        m = m.at[2].set(upd[i]['bu2p'])
        if i >= 1:
            m = m.at[3, 0:SPH].set(wn[i - 1])
        if i < 2:
            m = m.at[4, 0:D].set(msg[i + 1]['b1'])
            m = m.at[5].set(msg[i + 1]['b2p'])
        return m

    bv0 = jnp.zeros((8, W), f32).at[0, 0:D].set(msg[0]['b1'])
    bv0 = bv0.at[1].set(msg[0]['b2p'])

    dve, dvf = _sc_dvec(pos_p, se, de, sf, df)
    rbfcf, rsh = _tc_edge_geom(dve)
    gcomb = _tc_full_geom(dvf, wcomb)
    xs, invs, vtab = _tc_embed_inv(an_p, emb_p, msg[0]['W1'], msg[0]['W2p'],
                                   bv0)
    xv = jnp.zeros((NP, SV), f32)

    xva = None
    nsph = None
    xvtab = None
    for i in range(3):
        flts, etab = _tc_filt(rbfcf, rsh, msg[i]['Wrp'])
        sagg = _sc_msg(invs, flts, se, de)
        vagg = _sc_pair(vtab, etab, se, de)
        has_sph = 0 if i == 0 else (1 if i == 1 else 2)
        emit_inv = i < 2
        outs = _tc_update(has_sph, emit_inv, xs, xv, sagg, vagg, nsph,
                          upd[i]['U'], upd[i]['V'], upd[i]['Wu1a'],
                          upd[i]['Wu2p'], mk_misc(i),
                          msg[i + 1]['W1'] if emit_inv else None,
                          msg[i + 1]['W2p'] if emit_inv else None,
                          xva)
        outs = list(outs)
        xs, xv = outs[0], outs[1]
        k = 2
        if has_sph:
            nsph = outs[k]; k += 1
        if emit_inv:
            invs = outs[k]; k += 1
            vtab = outs[k]; k += 1
        if has_sph == 2:
            xvtab = outs[k]
        if i == 1:
            xva = xv

    ea1 = _sc_pair(xvtab, gcomb, sf, df)
    ea2 = _sc_pair(xvtab, gcomb, df, df)
    out = _tc_final(xs, nsph, ea1, ea2, w1a, w1b, po['W2'], wep, mscf)
    return out[:N]


# speedup vs baseline: 2.7013x; 1.2782x over previous
import functools

import jax
import jax.numpy as jnp
from jax import lax
from jax.experimental import pallas as pl
from jax.experimental.pallas import tpu as pltpu
from jax.experimental.pallas import tpu_sc as plsc

D = 128
NB = 20
CUT = 5.0
SPH = 9
SV = 16
W = 160
RW = 32
HID = 64
BLK = 32
N = 10000
NP = 10240
E = 160000
EP = 163840
CH = 64
NTILES = 32
EPT = EP // NTILES
NIT = EPT // CH
RPS = NP // 16
NR = 10112
RPS_S = NR // 16
BN = 1024
BE = 4096

@functools.cache
def _mesh():
    return plsc.VectorSubcoreMesh(core_axis_name="c", subcore_axis_name="s")


def _sig(x):
    return 1.0 / (1.0 + jnp.exp(-x))


def _silu(x):
    return x * _sig(x)


_SCHUNKS = [(k * CH, CH) for k in range(RPS_S // CH)] + [
    ((RPS_S // CH) * CH, RPS_S - (RPS_S // CH) * CH)]


def _sc_dvec_body(pos_h, se_h, de_h, sf_h, df_h, dve_h, dvf_h,
                  spos, ia, ib, pa, dv):
    s = lax.axis_index("s")
    wid = s * 2 + lax.axis_index("c")
    for off, ln in _SCHUNKS:
        pltpu.sync_copy(pos_h.at[pl.ds(s * RPS_S + off, ln)],
                        pa.at[pl.ds(0, ln)])
        pltpu.sync_copy(pa.at[pl.ds(0, ln)],
                        spos.at[pl.ds(s * RPS_S + off, ln)])
    plsc.subcore_barrier()

    def do_list(src_h, dst_h, out_h):
        def it_body(it, _):
            base = wid * EPT + it * CH
            pltpu.sync_copy(src_h.at[pl.ds(base, CH)], ia)
            pltpu.sync_copy(dst_h.at[pl.ds(base, CH)], ib)
            pltpu.sync_copy(spos.at[ia], pa)

            def row1(r, c):
                dv[r, :] = -pa[r, :]
                return c
            lax.fori_loop(0, CH, row1, 0)
            pltpu.sync_copy(spos.at[ib], pa)

            def row2(r, c):
                dv[r, :] = dv[r, :] + pa[r, :]
                return c
            lax.fori_loop(0, CH, row2, 0)
            pltpu.sync_copy(dv, out_h.at[pl.ds(base, CH)])
            return _
        lax.fori_loop(0, NIT, it_body, 0)

    do_list(se_h, de_h, dve_h)
    do_list(sf_h, df_h, dvf_h)


def _sc_dvec(pos_p, se, de, sf, df):
    fn = pl.kernel(
        _sc_dvec_body,
        out_type=[jax.ShapeDtypeStruct((EP, SV), jnp.float32),
                  jax.ShapeDtypeStruct((EP, SV), jnp.float32)],
        mesh=_mesh(),
        scratch_types=[pltpu.VMEM_SHARED((NR, SV), jnp.float32),
                       pltpu.VMEM((CH,), jnp.int32),
                       pltpu.VMEM((CH,), jnp.int32),
                       pltpu.VMEM((CH, SV), jnp.float32),
                       pltpu.VMEM((CH, SV), jnp.float32)],
    )
    return fn(pos_p, se, de, sf, df)


def _sc_msg_body(inv_h, flts_h, src_h, dst_h, sagg_h,
                 accs, iv_s0, iv_d0, iv_s1, iv_d1, ginv0, gflt0, ginv1,
                 gflt1, ms, sem0, sem1):
    c = lax.axis_index("c")
    s = lax.axis_index("s")
    wid = s * 2 + c
    slots = ((iv_s0, iv_d0, ginv0, gflt0, sem0),
             (iv_s1, iv_d1, ginv1, gflt1, sem1))

    def zrow(r, carry):
        for j in range(8):
            ms[r, pl.ds(16 * j, 16)] = jnp.zeros((16,), jnp.float32)
        return carry
    lax.fori_loop(0, CH, zrow, 0)
    for off, ln in _SCHUNKS:
        pltpu.sync_copy(ms.at[pl.ds(0, ln)],
                        accs.at[pl.ds(s * RPS_S + off, ln)])
    plsc.subcore_barrier()

    def issue(slot, base):
        ivs, ivd, gi, gf, sem = slot
        pltpu.sync_copy(src_h.at[pl.ds(base, CH)], ivs)
        pltpu.sync_copy(dst_h.at[pl.ds(base, CH)], ivd)
        pltpu.async_copy(inv_h.at[ivs], gi, sem)
        pltpu.async_copy(flts_h.at[pl.ds(base, CH)], gf, sem)

    def wait(slot, base):
        ivs, ivd, gi, gf, sem = slot
        pltpu.make_async_copy(inv_h.at[ivs], gi, sem).wait()
        pltpu.make_async_copy(flts_h.at[pl.ds(base, CH)], gf, sem).wait()

    issue(slots[0], wid * EPT)
    issue(slots[1], wid * EPT + CH)

    def it_body(k, _):
        for b in range(2):
            g = 2 * k + b
            base = wid * EPT + g * CH
            slot = slots[b]
            ivs, ivd, gi, gf, sem = slot
            wait(slot, base)

            def row(r, carry):
                for rr in range(2):
                    for j in range(8):
                        ms[2 * r + rr, pl.ds(16 * j, 16)] = (
                            gi[2 * r + rr, pl.ds(16 * j, 16)]
                            * gf[2 * r + rr, pl.ds(16 * j, 16)])
                return carry
            lax.fori_loop(0, CH // 2, row, 0)
            pltpu.sync_copy(ms, accs.at[ivd], add=True)
            nxt = jnp.minimum(g + 2, NIT - 1)
            issue(slot, wid * EPT + nxt * CH)
        return _
    lax.fori_loop(0, NIT // 2, it_body, 0)
    for b in range(2):
        ivs, ivd, gi, gf, sem = slots[b]
        last = wid * EPT + (NIT - 1) * CH
        wait(slots[b], last)
    plsc.subcore_barrier()
    for off, ln in _SCHUNKS:
        pltpu.sync_copy(accs.at[pl.ds(s * RPS_S + off, ln)],
                        ms.at[pl.ds(0, ln)])
        pltpu.sync_copy(ms.at[pl.ds(0, ln)],
                        sagg_h.at[c, pl.ds(s * RPS_S + off, ln)])


@functools.cache
def _sc_msg_fn():
    return pl.kernel(
        _sc_msg_body,
        out_type=[jax.ShapeDtypeStruct((2, NP, D), jnp.float32)],
        mesh=_mesh(),
        scratch_types=[pltpu.VMEM_SHARED((NR, D), jnp.float32)]
                      + [pltpu.VMEM((CH,), jnp.int32)] * 2
                      + [pltpu.VMEM((CH,), jnp.int32)] * 2
                      + [pltpu.VMEM((CH, D), jnp.float32)] * 2
                      + [pltpu.VMEM((CH, D), jnp.float32)] * 2
                      + [pltpu.VMEM((CH, D), jnp.float32)]
                      + [pltpu.SemaphoreType.DMA] * 2,
    )


def _sc_msg(invs, flts, src, dst):
    return _sc_msg_fn()(invs, flts, src, dst)[0]


def _sc_pair_body(tab_h, gc_h, gi_h, si_h, agg_h,
                  acc, ig0, is0, ig1, is1, gt0, gc0, gt1, gc1, ef,
                  sem0, sem1):
    c = lax.axis_index("c")
    s = lax.axis_index("s")
    wid = s * 2 + c
    slots = ((ig0, is0, gt0, gc0, sem0), (ig1, is1, gt1, gc1, sem1))

    def zrow(r, carry):
        ef[r, :] = jnp.zeros((16,), jnp.float32)
        return carry
    lax.fori_loop(0, CH, zrow, 0)
    for off, ln in _SCHUNKS:
        pltpu.sync_copy(ef.at[pl.ds(0, ln)],
                        acc.at[pl.ds(s * RPS_S + off, ln)])
    plsc.subcore_barrier()

    def issue(slot, base):
        ivg, ivs, gt, gcv, sem = slot
        pltpu.sync_copy(gi_h.at[pl.ds(base, CH)], ivg)
        pltpu.sync_copy(si_h.at[pl.ds(base, CH)], ivs)
        pltpu.async_copy(tab_h.at[ivg], gt, sem)
        pltpu.async_copy(gc_h.at[pl.ds(base, CH)], gcv, sem)

    def wait(slot, base):
        ivg, ivs, gt, gcv, sem = slot
        pltpu.make_async_copy(tab_h.at[ivg], gt, sem).wait()
        pltpu.make_async_copy(gc_h.at[pl.ds(base, CH)], gcv, sem).wait()

    issue(slots[0], wid * EPT)
    issue(slots[1], wid * EPT + CH)

    def it_body(k, _):
        for b in range(2):
            g = 2 * k + b
            base = wid * EPT + g * CH
            slot = slots[b]
            ivg, ivs, gt, gcv, sem = slot
            wait(slot, base)

            def row(r, carry):
                for rr in range(2):
                    ef[2 * r + rr, :] = (
                        gcv[2 * r + rr, pl.ds(0, 16)]
                        * gt[2 * r + rr, pl.ds(0, 16)]
                        + gcv[2 * r + rr, pl.ds(16, 16)]
                        * gt[2 * r + rr, pl.ds(16, 16)])
                return carry
            lax.fori_loop(0, CH // 2, row, 0)
            pltpu.sync_copy(ef, acc.at[ivs], add=True)
            nxt = jnp.minimum(g + 2, NIT - 1)
            issue(slot, wid * EPT + nxt * CH)
        return _
    lax.fori_loop(0, NIT // 2, it_body, 0)
    last = wid * EPT + (NIT - 1) * CH
    for b in range(2):
        wait(slots[b], last)
    plsc.subcore_barrier()
    for off, ln in _SCHUNKS:
        pltpu.sync_copy(acc.at[pl.ds(s * RPS_S + off, ln)],
                        ef.at[pl.ds(0, ln)])
        pltpu.sync_copy(ef.at[pl.ds(0, ln)],
                        agg_h.at[c, pl.ds(s * RPS_S + off, ln)])


@functools.cache
def _sc_pair_fn():
    return pl.kernel(
        _sc_pair_body,
        out_type=[jax.ShapeDtypeStruct((2, NP, SV), jnp.float32)],
        mesh=_mesh(),
        scratch_types=[pltpu.VMEM_SHARED((NR, SV), jnp.float32)]
                      + [pltpu.VMEM((CH,), jnp.int32)] * 4
                      + [pltpu.VMEM((CH, D), jnp.float32),
                         pltpu.VMEM((CH, RW), jnp.float32),
                         pltpu.VMEM((CH, D), jnp.float32),
                         pltpu.VMEM((CH, RW), jnp.float32),
                         pltpu.VMEM((CH, SV), jnp.float32)]
                      + [pltpu.SemaphoreType.DMA] * 2,
    )


def _sc_pair(tab, gc, gidx, sidx):
    return _sc_pair_fn()(tab, gc, gidx, sidx)[0]


def _tc(body, grid, in_specs, out_specs, out_shapes):
    single = not isinstance(out_shapes, (list, tuple))
    if single:
        out_shapes = [out_shapes]
    fn = pl.pallas_call(
        body, grid=grid, in_specs=list(in_specs),
        out_specs=list(out_specs), out_shape=list(out_shapes))
    if single:
        return lambda *a: fn(*a)[0]
    return fn


def _full(shape):
    return pl.BlockSpec(shape, lambda i: (0,) * len(shape))


def _rows(bs, width):
    return pl.BlockSpec((bs, width), lambda i: (i, 0))


def _embed_inv_body(an_ref, emb_ref, w1_ref, w2_ref, bv_ref,
                    xs_ref, invs_ref, vtab_ref):
    an = an_ref[...]
    lanes = lax.broadcasted_iota(jnp.int32, (BN, D), 1)
    oh = (an == lanes).astype(jnp.float32)
    xs = jnp.dot(oh, emb_ref[...], preferred_element_type=jnp.float32)
    xs_ref[...] = xs
    b1 = bv_ref[0:1, 0:D]
    b2 = bv_ref[1:2, :]
    t = _silu(jnp.dot(xs, w1_ref[...], preferred_element_type=jnp.float32) + b1)
    inv = jnp.dot(t, w2_ref[...], preferred_element_type=jnp.float32) + b2
    invs_ref[...] = inv[:, 0:D]
    vtab_ref[...] = jnp.concatenate(
        [jnp.zeros((BN, SV), jnp.float32), inv[:, 144:160],
         jnp.zeros((BN, 96), jnp.float32)], axis=1)


def _tc_embed_inv(an_p, emb_p, w1, w2p, bv):
    return _tc(
        _embed_inv_body, (NP // BN,),
        [_rows(BN, 1), _full((D, D)), _full((D, D)), _full((D, W)),
         _full((8, W))],
        [_rows(BN, D), _rows(BN, D), _rows(BN, D)],
        (jax.ShapeDtypeStruct((NP, D), jnp.float32),
         jax.ShapeDtypeStruct((NP, D), jnp.float32),
         jax.ShapeDtypeStruct((NP, D), jnp.float32)),
    )(an_p, emb_p, w1, w2p, bv)


def _geom_common(dv):
    x = dv[:, 0:1]
    y = dv[:, 1:2]
    z = dv[:, 2:3]
    d = jnp.sqrt(x * x + y * y + z * z + 1e-12)
    nlane = (lax.broadcasted_iota(jnp.int32, (dv.shape[0], RW), 1)
             .astype(jnp.float32) + 1.0)
    rbf = jnp.sin(nlane * (jnp.pi / CUT) * d) / d
    dmin = jnp.minimum(d, CUT)
    fcut = 0.5 * (jnp.cos(jnp.pi * dmin / CUT) + 1.0) * (d < CUT)
    return x / d, y / d, z / d, rbf, fcut


def _edge_geom_body(dv_ref, rbfcf_ref, rsh_ref):
    pid = pl.program_id(0)
    dv = dv_ref[...]
    ux, uy, uz, rbf, fcut = _geom_common(dv)
    gid = pid * BE + lax.broadcasted_iota(jnp.int32, (BE, 1), 0)
    mask = (gid < E).astype(jnp.float32)
    lanes = lax.broadcasted_iota(jnp.int32, (BE, RW), 1)
    fc = fcut * mask
    rbfcf = jnp.where(lanes < NB, rbf * fc, jnp.where(lanes == NB, fc, 0.0))
    rbfcf_ref[...] = rbfcf
    l16 = lax.broadcasted_iota(jnp.int32, (BE, SV), 1)
    r = jnp.where(l16 == 0, 1.0, 0.0)
    for i, t in enumerate([ux, uy, uz, ux * uy, uy * uz,
                           3.0 * uz * uz - 1.0, ux * uz, ux * ux - uy * uy]):
        r = jnp.where(l16 == i + 1, t, r)
    rsh_ref[...] = r


def _tc_edge_geom(dve):
    return _tc(
        _edge_geom_body, (EP // BE,),
        [_rows(BE, SV)],
        [_rows(BE, RW), _rows(BE, SV)],
        (jax.ShapeDtypeStruct((EP, RW), jnp.float32),
         jax.ShapeDtypeStruct((EP, SV), jnp.float32)),
    )(dve)


def _full_geom_body(dv_ref, wc_ref, gc_ref):
    pid = pl.program_id(0)
    dv = dv_ref[...]
    _, _, _, rbf, fcut = _geom_common(dv)
    gid = pid * BE + lax.broadcasted_iota(jnp.int32, (BE, 1), 0)
    mask = (gid < E).astype(jnp.float32)
    lanes = lax.broadcasted_iota(jnp.int32, (BE, RW), 1)
    frbf = jnp.where(lanes < NB, rbf * fcut * mask, 0.0)
    gc_ref[...] = jnp.dot(frbf, wc_ref[...],
                          preferred_element_type=jnp.float32)


def _tc_full_geom(dvf, wcomb):
    return _tc(
        _full_geom_body, (EP // BE,),
        [_rows(BE, SV), _full((RW, RW))],
        [_rows(BE, RW)],
        jax.ShapeDtypeStruct((EP, RW), jnp.float32),
    )(dvf, wcomb)


def _filt_body(rb_ref, rsh_ref, wr_ref, flts_ref, etab_ref):
    flt = jnp.dot(rb_ref[...], wr_ref[...], preferred_element_type=jnp.float32)
    flts_ref[...] = flt[:, 0:D]
    etab_ref[...] = jnp.concatenate(
        [flt[:, D:D + SV], flt[:, D + SV:W] * rsh_ref[...]], axis=1)


def _tc_filt(rbfcf, rsh, wrp):
    return _tc(
        _filt_body, (EP // BE,),
        [_rows(BE, RW), _rows(BE, SV), _full((RW, W))],
        [_rows(BE, D), _rows(BE, RW)],
        (jax.ShapeDtypeStruct((EP, D), jnp.float32),
         jax.ShapeDtypeStruct((EP, RW), jnp.float32)),
    )(rbfcf, rsh, wrp)


def _make_update_body(has_sph, emit_inv):
    def body(*refs):
        i = 0
        xs_ref = refs[i]; i += 1
        xv_ref = refs[i]; i += 1
        sagg_ref = refs[i]; i += 1
        vagg_ref = refs[i]; i += 1
        if has_sph == 2:
            nin_ref = refs[i]; i += 1
            xva_ref = refs[i]; i += 1
        up_ref = refs[i]; i += 1
        vp_ref = refs[i]; i += 1
        wu1_ref = refs[i]; i += 1
        wu2_ref = refs[i]; i += 1
        msc_ref = refs[i]; i += 1
        if emit_inv:
            w1n_ref = refs[i]; i += 1
            w2n_ref = refs[i]; i += 1
        xs_o = refs[i]; i += 1
        xv_o = refs[i]; i += 1
        if has_sph:
            nsph_o = refs[i]; i += 1
        if emit_inv:
            invs_o = refs[i]; i += 1
            vtab_o = refs[i]; i += 1
        if has_sph == 2:
            xvt_o = refs[i]; i += 1

        f32 = jnp.float32
        xs1 = xs_ref[...] + sagg_ref[0] + sagg_ref[1]
        mu = jnp.mean(xs1, axis=-1, keepdims=True)
        var = jnp.mean((xs1 - mu) ** 2, axis=-1, keepdims=True)
        xs1 = (xs1 - mu) / jnp.sqrt(var + 1e-5)
        xv1 = xv_ref[...] + vagg_ref[0] + vagg_ref[1]
        uv = jnp.dot(xv1, up_ref[...], preferred_element_type=f32)
        vv = jnp.dot(xv1, vp_ref[...], preferred_element_type=f32)
        vnorm = jnp.sqrt(jnp.sum(vv * vv, axis=-1, keepdims=True) + 1e-12)
        wu1b = msc_ref[0:1, 0:D]
        bu1 = msc_ref[1:2, 0:D]
        bu2 = msc_ref[2:3, :]
        t = _silu(jnp.dot(xs1, wu1_ref[...], preferred_element_type=f32)
                  + vnorm * wu1b + bu1)
        a = jnp.dot(t, wu2_ref[...], preferred_element_type=f32) + bu2
        dots = jnp.sum(uv * vv, axis=-1, keepdims=True)
        xs2 = xs1 + a[:, 0:D] + a[:, D:D + 1] * dots
        xv2 = xv1 + a[:, 144:160] * uv
        xs_o[...] = xs2
        xv_o[...] = xv2
        if has_sph:
            wn = msc_ref[3:4, 0:SV]
            ns = xv2 * wn
            if has_sph == 2:
                ns = ns + nin_ref[...]
            nsph_o[...] = ns
        if emit_inv:
            b1n = msc_ref[4:5, 0:D]
            b2n = msc_ref[5:6, :]
            tn = _silu(jnp.dot(xs2, w1n_ref[...], preferred_element_type=f32)
                       + b1n)
            invf = jnp.dot(tn, w2n_ref[...], preferred_element_type=f32) + b2n
            invs_o[...] = invf[:, 0:D]
            vtab_o[...] = jnp.concatenate(
                [invf[:, D:D + SV] * xv2, invf[:, D + SV:W],
                 jnp.zeros((BN, 96), f32)], axis=1)
        if has_sph == 2:
            xvt_o[...] = jnp.concatenate(
                [xva_ref[...], xv2, jnp.zeros((BN, 96), f32)], axis=1)
    return body


def _tc_update(has_sph, emit_inv, xs, xv, sagg, vagg, nin, up, vp, wu1, wu2,
               msc, w1n, w2n, xva=None):
    in_arrays = [xs, xv, sagg, vagg]
    in_specs = [_rows(BN, D), _rows(BN, SV),
                pl.BlockSpec((2, BN, D), lambda i: (0, i, 0)),
                pl.BlockSpec((2, BN, SV), lambda i: (0, i, 0))]
    if has_sph == 2:
        in_arrays += [nin, xva]
        in_specs += [_rows(BN, SV), _rows(BN, SV)]
    in_arrays += [up, vp, wu1, wu2, msc]
    in_specs += [_full((SV, SV)), _full((SV, SV)), _full((D, D)),
                 _full((D, W)), _full((8, W))]
    if emit_inv:
        in_arrays += [w1n, w2n]
        in_specs += [_full((D, D)), _full((D, W))]
    out_specs = [_rows(BN, D), _rows(BN, SV)]
    out_shapes = [jax.ShapeDtypeStruct((NP, D), jnp.float32),
                  jax.ShapeDtypeStruct((NP, SV), jnp.float32)]
    if has_sph:
        out_specs.append(_rows(BN, SV))
        out_shapes.append(jax.ShapeDtypeStruct((NP, SV), jnp.float32))
    if emit_inv:
        out_specs += [_rows(BN, D), _rows(BN, D)]
        out_shapes += [jax.ShapeDtypeStruct((NP, D), jnp.float32),
                       jax.ShapeDtypeStruct((NP, D), jnp.float32)]
    if has_sph == 2:
        out_specs.append(_rows(BN, D))
        out_shapes.append(jax.ShapeDtypeStruct((NP, D), jnp.float32))
    return _tc(_make_update_body(has_sph, emit_inv), (NP // BN,),
               in_specs, out_specs, tuple(out_shapes))(*in_arrays)


def _final_body(xs_ref, ns_ref, ea_ref, eb_ref, w1a_ref, w1b_ref, w2_ref,
                wep_ref, msc_ref, out_ref):
    f32 = jnp.float32
    b1 = msc_ref[0:1, :]
    b2 = msc_ref[1:2, 0:BLK]
    e9 = ea_ref[0] + ea_ref[1] + eb_ref[0] + eb_ref[1]
    nh = _silu(jnp.dot(xs_ref[...], w1a_ref[...], preferred_element_type=f32)
               + jnp.dot(ns_ref[...], w1b_ref[...], preferred_element_type=f32)
               + b1)
    out_ref[...] = (jnp.dot(nh, w2_ref[...], preferred_element_type=f32) + b2
                    + jnp.dot(e9, wep_ref[...], preferred_element_type=f32))


def _tc_final(xs, nsph, eagg, ebgg, w1a, w1b, w2, wep, msc):
    return _tc(
        _final_body, (NP // BN,),
        [_rows(BN, D), _rows(BN, SV),
         pl.BlockSpec((2, BN, SV), lambda i: (0, i, 0)),
         pl.BlockSpec((2, BN, SV), lambda i: (0, i, 0)),
         _full((D, HID)), _full((SV, HID)), _full((HID, BLK)),
         _full((SV, BLK)), _full((8, HID))],
        [_rows(BN, BLK)],
        jax.ShapeDtypeStruct((NP, BLK), jnp.float32),
    )(xs, nsph, eagg, ebgg, w1a, w1b, w2, wep, msc)


def _pad_cols(w, b):
    fi = w.shape[0]
    wp = jnp.zeros((fi, W), jnp.float32)
    wp = wp.at[:, 0:D].set(w[:, 0:D])
    wp = wp.at[:, D:D + SPH].set(w[:, D:D + SPH])
    wp = wp.at[:, 144:144 + SPH].set(w[:, D + SPH:D + 2 * SPH])
    bp = jnp.zeros((W,), jnp.float32)
    bp = bp.at[0:D].set(b[0:D])
    bp = bp.at[D:D + SPH].set(b[D:D + SPH])
    bp = bp.at[144:144 + SPH].set(b[D + SPH:D + 2 * SPH])
    return wp, bp


def kernel(at_no, pos, edge_index, edge_index_full, params):
    f32 = jnp.float32
    i32 = jnp.int32

    an_p = jnp.zeros((NP, 1), i32).at[:N, 0].set(at_no.astype(i32))
    pos_p = jnp.zeros((NR, SV), f32).at[:N, :3].set(pos)
    npad = EP - E
    spread = ((jnp.arange(npad, dtype=i32) * 37) % (N - 1)).astype(i32)
    def padidx(a):
        return jnp.concatenate([a.astype(i32), spread])
    se, de = padidx(edge_index[0]), padidx(edge_index[1])
    sf, df = padidx(edge_index_full[0]), padidx(edge_index_full[1])

    emb_p = jnp.zeros((D, D), f32).at[:100].set(params['embed'])
    msg, upd = [], []
    for i in range(3):
        pm = params['msg%d' % i]
        w2p, b2p = _pad_cols(pm['W2'], pm['b2'])
        wrp_, brp = _pad_cols(pm['Wr'], pm['br'])
        wrp = jnp.zeros((RW, W), f32).at[0:NB].set(wrp_).at[NB].set(brp)
        msg.append({'W1': pm['W1'], 'b1': pm['b1'], 'W2p': w2p, 'b2p': b2p,
                    'Wrp': wrp})
        pu = params['upd%d' % i]
        up = jnp.zeros((SV, SV), f32).at[:SPH, :SPH].set(pu['U'])
        vp = jnp.zeros((SV, SV), f32).at[:SPH, :SPH].set(pu['V'])
        wu2 = pu['Wu2']
        wu2p = jnp.zeros((D, W), f32)
        wu2p = wu2p.at[:, 0:D].set(wu2[:, 0:D])
        wu2p = wu2p.at[:, D:D + 1].set(wu2[:, D:D + 1])
        wu2p = wu2p.at[:, 144:144 + SPH].set(wu2[:, D + 1:D + 1 + SPH])
        bu2p = jnp.zeros((W,), f32)
        bu2p = bu2p.at[0:D].set(pu['bu2'][0:D])
        bu2p = bu2p.at[D:D + 1].set(pu['bu2'][D:D + 1])
        bu2p = bu2p.at[144:144 + SPH].set(pu['bu2'][D + 1:D + 1 + SPH])
        upd.append({'U': up, 'V': vp, 'Wu1a': pu['Wu1'][:D, :],
                    'wu1b': pu['Wu1'][D, :], 'bu1': pu['bu1'], 'bu2p': bu2p,
                    'Wu2p': wu2p})
    wn = [params['mat%d' % j]['wn'] for j in range(2)]
    wcomb = jnp.zeros((RW, RW), f32)
    wcomb = wcomb.at[0:NB, 0:SPH].set(params['mat0']['We'])
    wcomb = wcomb.at[0:NB, SV:SV + SPH].set(params['mat1']['We'])
    po = params['out']
    w1a = po['W1'][:D, :]
    w1b = jnp.zeros((SV, HID), f32).at[:SPH].set(po['W1'][D:D + SPH, :])
    wep = jnp.zeros((SV, BLK), f32).at[:SPH].set(po['We'])
    mscf = jnp.zeros((8, HID), f32).at[0].set(po['b1'])
    mscf = mscf.at[1, 0:BLK].set(po['b2'])

    def mk_misc(i):
        m = jnp.zeros((8, W), f32)
        m = m.at[0, 0:D].set(upd[i]['wu1b'])
        m = m.at[1, 0:D].set(upd[i]['bu1'])
        m = m.at[2].set(upd[i]['bu2p'])
        if i >= 1:
            m = m.at[3, 0:SPH].set(wn[i - 1])
        if i < 2:
            m = m.at[4, 0:D].set(msg[i + 1]['b1'])
            m = m.at[5].set(msg[i + 1]['b2p'])
        return m

    bv0 = jnp.zeros((8, W), f32).at[0, 0:D].set(msg[0]['b1'])
    bv0 = bv0.at[1].set(msg[0]['b2p'])

    dve, dvf = _sc_dvec(pos_p, se, de, sf, df)
    rbfcf, rsh = _tc_edge_geom(dve)
    gcomb = _tc_full_geom(dvf, wcomb)
    xs, invs, vtab = _tc_embed_inv(an_p, emb_p, msg[0]['W1'], msg[0]['W2p'],
                                   bv0)
    xv = jnp.zeros((NP, SV), f32)

    xva = None
    nsph = None
    xvtab = None
    for i in range(3):
        flts, etab = _tc_filt(rbfcf, rsh, msg[i]['Wrp'])
        sagg = _sc_msg(invs, flts, se, de)
        vagg = _sc_pair(vtab, etab, se, de)
        has_sph = 0 if i == 0 else (1 if i == 1 else 2)
        emit_inv = i < 2
        outs = _tc_update(has_sph, emit_inv, xs, xv, sagg, vagg, nsph,
                          upd[i]['U'], upd[i]['V'], upd[i]['Wu1a'],
                          upd[i]['Wu2p'], mk_misc(i),
                          msg[i + 1]['W1'] if emit_inv else None,
                          msg[i + 1]['W2p'] if emit_inv else None,
                          xva)
        outs = list(outs)
        xs, xv = outs[0], outs[1]
        k = 2
        if has_sph:
            nsph = outs[k]; k += 1
        if emit_inv:
            invs = outs[k]; k += 1
            vtab = outs[k]; k += 1
        if has_sph == 2:
            xvtab = outs[k]
        if i == 1:
            xva = xv

    ea1 = _sc_pair(xvtab, gcomb, sf, df)
    ea2 = _sc_pair(xvtab, gcomb, df, df)
    out = _tc_final(xs, nsph, ea1, ea2, w1a, w1b, po['W2'], wep, mscf)
    return out[:N]


# speedup vs baseline: 2.8641x; 1.0603x over previous
import functools

import jax
import jax.numpy as jnp
from jax import lax
from jax.experimental import pallas as pl
from jax.experimental.pallas import tpu as pltpu
from jax.experimental.pallas import tpu_sc as plsc

D = 128
NB = 20
CUT = 5.0
SPH = 9
SV = 16
W = 160
RW = 32
HID = 64
BLK = 32
N = 10000
NP = 10240
E = 160000
EP = 163840
CH = 64
NTILES = 32
EPT = EP // NTILES
NIT = EPT // CH
RPS = NP // 16
NR = 10112
RPS_S = NR // 16
BN = 1024
BE = 4096

@functools.cache
def _mesh():
    return plsc.VectorSubcoreMesh(core_axis_name="c", subcore_axis_name="s")


def _sig(x):
    return 1.0 / (1.0 + jnp.exp(-x))


def _silu(x):
    return x * _sig(x)


_SCHUNKS = [(k * CH, CH) for k in range(RPS_S // CH)] + [
    ((RPS_S // CH) * CH, RPS_S - (RPS_S // CH) * CH)]


def _sc_dvec_body(pos_h, se_h, de_h, sf_h, df_h, dve_h, dvf_h,
                  spos, ia, ib, pa, dv):
    s = lax.axis_index("s")
    wid = s * 2 + lax.axis_index("c")
    for off, ln in _SCHUNKS:
        pltpu.sync_copy(pos_h.at[pl.ds(s * RPS_S + off, ln)],
                        pa.at[pl.ds(0, ln)])
        pltpu.sync_copy(pa.at[pl.ds(0, ln)],
                        spos.at[pl.ds(s * RPS_S + off, ln)])
    plsc.subcore_barrier()

    def do_list(src_h, dst_h, out_h):
        def it_body(it, _):
            base = wid * EPT + it * CH
            pltpu.sync_copy(src_h.at[pl.ds(base, CH)], ia)
            pltpu.sync_copy(dst_h.at[pl.ds(base, CH)], ib)
            pltpu.sync_copy(spos.at[ia], pa)

            def row1(r, c):
                dv[r, :] = -pa[r, :]
                return c
            lax.fori_loop(0, CH, row1, 0)
            pltpu.sync_copy(spos.at[ib], pa)

            def row2(r, c):
                dv[r, :] = dv[r, :] + pa[r, :]
                return c
            lax.fori_loop(0, CH, row2, 0)
            pltpu.sync_copy(dv, out_h.at[pl.ds(base, CH)])
            return _
        lax.fori_loop(0, NIT, it_body, 0)

    do_list(se_h, de_h, dve_h)
    do_list(sf_h, df_h, dvf_h)


def _sc_dvec(pos_p, se, de, sf, df):
    fn = pl.kernel(
        _sc_dvec_body,
        out_type=[jax.ShapeDtypeStruct((EP, SV), jnp.float32),
                  jax.ShapeDtypeStruct((EP, SV), jnp.float32)],
        mesh=_mesh(),
        scratch_types=[pltpu.VMEM_SHARED((NR, SV), jnp.float32),
                       pltpu.VMEM((CH,), jnp.int32),
                       pltpu.VMEM((CH,), jnp.int32),
                       pltpu.VMEM((CH, SV), jnp.float32),
                       pltpu.VMEM((CH, SV), jnp.float32)],
    )
    return fn(pos_p, se, de, sf, df)


def _sc_msg_body(inv_h, flts_h, src_h, dst_h, sagg_h,
                 accs, iv_s0, iv_d0, iv_s1, iv_d1, ivds, ginv0, gflt0,
                 ginv1, gflt1, ms, sem0, sem1, semi0, semi1):
    c = lax.axis_index("c")
    s = lax.axis_index("s")
    wid = s * 2 + c
    slots = ((iv_s0, iv_d0, ginv0, gflt0, sem0, semi0),
             (iv_s1, iv_d1, ginv1, gflt1, sem1, semi1))

    def zrow(r, carry):
        for j in range(8):
            ms[r, pl.ds(16 * j, 16)] = jnp.zeros((16,), jnp.float32)
        return carry
    lax.fori_loop(0, CH, zrow, 0)
    for off, ln in _SCHUNKS:
        pltpu.sync_copy(ms.at[pl.ds(0, ln)],
                        accs.at[pl.ds(s * RPS_S + off, ln)])
    plsc.subcore_barrier()

    def issue_idx(slot, base):
        ivs, ivd, gi, gf, sem, semi = slot
        pltpu.async_copy(src_h.at[pl.ds(base, CH)], ivs, semi)
        pltpu.async_copy(dst_h.at[pl.ds(base, CH)], ivd, semi)

    def wait_idx(slot, base):
        ivs, ivd, gi, gf, sem, semi = slot
        pltpu.make_async_copy(src_h.at[pl.ds(base, CH)], ivs, semi).wait()
        pltpu.make_async_copy(dst_h.at[pl.ds(base, CH)], ivd, semi).wait()

    def issue_data(slot, base):
        ivs, ivd, gi, gf, sem, semi = slot
        pltpu.async_copy(inv_h.at[ivs], gi, sem)
        pltpu.async_copy(flts_h.at[pl.ds(base, CH)], gf, sem)

    def wait_data(slot, base):
        ivs, ivd, gi, gf, sem, semi = slot
        pltpu.make_async_copy(inv_h.at[ivs], gi, sem).wait()
        pltpu.make_async_copy(flts_h.at[pl.ds(base, CH)], gf, sem).wait()

    issue_idx(slots[0], wid * EPT)
    issue_idx(slots[1], wid * EPT + CH)
    wait_idx(slots[0], wid * EPT)
    issue_data(slots[0], wid * EPT)
    wait_idx(slots[1], wid * EPT + CH)
    issue_data(slots[1], wid * EPT + CH)

    def it_body(k, _):
        for b in range(2):
            g = 2 * k + b
            base = wid * EPT + g * CH
            slot = slots[b]
            ivs, ivd, gi, gf, sem, semi = slot
            wait_data(slot, base)
            for q in range(CH // 16):
                ivds[pl.ds(16 * q, 16)] = ivd[pl.ds(16 * q, 16)]
            nxt = jnp.minimum(g + 2, NIT - 1)
            nbase = wid * EPT + nxt * CH
            issue_idx(slot, nbase)

            def row(r, carry):
                for rr in range(2):
                    for j in range(8):
                        ms[2 * r + rr, pl.ds(16 * j, 16)] = (
                            gi[2 * r + rr, pl.ds(16 * j, 16)]
                            * gf[2 * r + rr, pl.ds(16 * j, 16)])
                return carry
            lax.fori_loop(0, CH // 2, row, 0)
            pltpu.sync_copy(ms, accs.at[ivds], add=True)
            wait_idx(slot, nbase)
            issue_data(slot, nbase)
        return _
    lax.fori_loop(0, NIT // 2, it_body, 0)
    last = wid * EPT + (NIT - 1) * CH
    for b in range(2):
        wait_data(slots[b], last)
    plsc.subcore_barrier()
    for off, ln in _SCHUNKS:
        pltpu.sync_copy(accs.at[pl.ds(s * RPS_S + off, ln)],
                        ms.at[pl.ds(0, ln)])
        pltpu.sync_copy(ms.at[pl.ds(0, ln)],
                        sagg_h.at[c, pl.ds(s * RPS_S + off, ln)])


@functools.cache
def _sc_msg_fn():
    return pl.kernel(
        _sc_msg_body,
        out_type=[jax.ShapeDtypeStruct((2, NP, D), jnp.float32)],
        mesh=_mesh(),
        scratch_types=[pltpu.VMEM_SHARED((NR, D), jnp.float32)]
                      + [pltpu.VMEM((CH,), jnp.int32)] * 5
                      + [pltpu.VMEM((CH, D), jnp.float32)] * 5
                      + [pltpu.SemaphoreType.DMA] * 4,
    )


def _sc_msg(invs, flts, src, dst):
    return _sc_msg_fn()(invs, flts, src, dst)[0]


def _sc_pair_body(tab_h, gc_h, gi_h, si_h, agg_h,
                  acc, ig0, is0, ig1, is1, isd, gt0, gc0, gt1, gc1, ef,
                  sem0, sem1, semi0, semi1):
    c = lax.axis_index("c")
    s = lax.axis_index("s")
    wid = s * 2 + c
    slots = ((ig0, is0, gt0, gc0, sem0, semi0),
             (ig1, is1, gt1, gc1, sem1, semi1))

    def zrow(r, carry):
        ef[r, :] = jnp.zeros((16,), jnp.float32)
        return carry
    lax.fori_loop(0, CH, zrow, 0)
    for off, ln in _SCHUNKS:
        pltpu.sync_copy(ef.at[pl.ds(0, ln)],
                        acc.at[pl.ds(s * RPS_S + off, ln)])
    plsc.subcore_barrier()

    def issue_idx(slot, base):
        ivg, ivs, gt, gcv, sem, semi = slot
        pltpu.async_copy(gi_h.at[pl.ds(base, CH)], ivg, semi)
        pltpu.async_copy(si_h.at[pl.ds(base, CH)], ivs, semi)

    def wait_idx(slot, base):
        ivg, ivs, gt, gcv, sem, semi = slot
        pltpu.make_async_copy(gi_h.at[pl.ds(base, CH)], ivg, semi).wait()
        pltpu.make_async_copy(si_h.at[pl.ds(base, CH)], ivs, semi).wait()

    def issue_data(slot, base):
        ivg, ivs, gt, gcv, sem, semi = slot
        pltpu.async_copy(tab_h.at[ivg], gt, sem)
        pltpu.async_copy(gc_h.at[pl.ds(base, CH)], gcv, sem)

    def wait_data(slot, base):
        ivg, ivs, gt, gcv, sem, semi = slot
        pltpu.make_async_copy(tab_h.at[ivg], gt, sem).wait()
        pltpu.make_async_copy(gc_h.at[pl.ds(base, CH)], gcv, sem).wait()

    issue_idx(slots[0], wid * EPT)
    issue_idx(slots[1], wid * EPT + CH)
    wait_idx(slots[0], wid * EPT)
    issue_data(slots[0], wid * EPT)
    wait_idx(slots[1], wid * EPT + CH)
    issue_data(slots[1], wid * EPT + CH)

    def it_body(k, _):
        for b in range(2):
            g = 2 * k + b
            base = wid * EPT + g * CH
            slot = slots[b]
            ivg, ivs, gt, gcv, sem, semi = slot
            wait_data(slot, base)
            for q in range(CH // 16):
                isd[pl.ds(16 * q, 16)] = ivs[pl.ds(16 * q, 16)]
            nxt = jnp.minimum(g + 2, NIT - 1)
            nbase = wid * EPT + nxt * CH
            issue_idx(slot, nbase)

            def row(r, carry):
                for rr in range(2):
                    ef[2 * r + rr, :] = (
                        gcv[2 * r + rr, pl.ds(0, 16)]
                        * gt[2 * r + rr, pl.ds(0, 16)]
                        + gcv[2 * r + rr, pl.ds(16, 16)]
                        * gt[2 * r + rr, pl.ds(16, 16)])
                return carry
            lax.fori_loop(0, CH // 2, row, 0)
            pltpu.sync_copy(ef, acc.at[isd], add=True)
            wait_idx(slot, nbase)
            issue_data(slot, nbase)
        return _
    lax.fori_loop(0, NIT // 2, it_body, 0)
    last = wid * EPT + (NIT - 1) * CH
    for b in range(2):
        wait_data(slots[b], last)
    plsc.subcore_barrier()
    for off, ln in _SCHUNKS:
        pltpu.sync_copy(acc.at[pl.ds(s * RPS_S + off, ln)],
                        ef.at[pl.ds(0, ln)])
        pltpu.sync_copy(ef.at[pl.ds(0, ln)],
                        agg_h.at[c, pl.ds(s * RPS_S + off, ln)])


@functools.cache
def _sc_pair_fn():
    return pl.kernel(
        _sc_pair_body,
        out_type=[jax.ShapeDtypeStruct((2, NP, SV), jnp.float32)],
        mesh=_mesh(),
        scratch_types=[pltpu.VMEM_SHARED((NR, SV), jnp.float32)]
                      + [pltpu.VMEM((CH,), jnp.int32)] * 5
                      + [pltpu.VMEM((CH, D), jnp.float32),
                         pltpu.VMEM((CH, RW), jnp.float32),
                         pltpu.VMEM((CH, D), jnp.float32),
                         pltpu.VMEM((CH, RW), jnp.float32),
                         pltpu.VMEM((CH, SV), jnp.float32)]
                      + [pltpu.SemaphoreType.DMA] * 4,
    )


def _sc_pair(tab, gc, gidx, sidx):
    return _sc_pair_fn()(tab, gc, gidx, sidx)[0]


def _tc(body, grid, in_specs, out_specs, out_shapes):
    single = not isinstance(out_shapes, (list, tuple))
    if single:
        out_shapes = [out_shapes]
    fn = pl.pallas_call(
        body, grid=grid, in_specs=list(in_specs),
        out_specs=list(out_specs), out_shape=list(out_shapes))
    if single:
        return lambda *a: fn(*a)[0]
    return fn


def _full(shape):
    return pl.BlockSpec(shape, lambda i: (0,) * len(shape))


def _rows(bs, width):
    return pl.BlockSpec((bs, width), lambda i: (i, 0))


def _embed_inv_body(an_ref, emb_ref, w1_ref, w2_ref, bv_ref,
                    xs_ref, invs_ref, vtab_ref):
    an = an_ref[...]
    lanes = lax.broadcasted_iota(jnp.int32, (BN, D), 1)
    oh = (an == lanes).astype(jnp.float32)
    xs = jnp.dot(oh, emb_ref[...], preferred_element_type=jnp.float32)
    xs_ref[...] = xs
    b1 = bv_ref[0:1, 0:D]
    b2 = bv_ref[1:2, :]
    t = _silu(jnp.dot(xs, w1_ref[...], preferred_element_type=jnp.float32) + b1)
    inv = jnp.dot(t, w2_ref[...], preferred_element_type=jnp.float32) + b2
    invs_ref[...] = inv[:, 0:D]
    vtab_ref[...] = jnp.concatenate(
        [jnp.zeros((BN, SV), jnp.float32), inv[:, 144:160],
         jnp.zeros((BN, 96), jnp.float32)], axis=1)


def _tc_embed_inv(an_p, emb_p, w1, w2p, bv):
    return _tc(
        _embed_inv_body, (NP // BN,),
        [_rows(BN, 1), _full((D, D)), _full((D, D)), _full((D, W)),
         _full((8, W))],
        [_rows(BN, D), _rows(BN, D), _rows(BN, D)],
        (jax.ShapeDtypeStruct((NP, D), jnp.float32),
         jax.ShapeDtypeStruct((NP, D), jnp.float32),
         jax.ShapeDtypeStruct((NP, D), jnp.float32)),
    )(an_p, emb_p, w1, w2p, bv)


def _geom_common(dv):
    x = dv[:, 0:1]
    y = dv[:, 1:2]
    z = dv[:, 2:3]
    d = jnp.sqrt(x * x + y * y + z * z + 1e-12)
    nlane = (lax.broadcasted_iota(jnp.int32, (dv.shape[0], RW), 1)
             .astype(jnp.float32) + 1.0)
    rbf = jnp.sin(nlane * (jnp.pi / CUT) * d) / d
    dmin = jnp.minimum(d, CUT)
    fcut = 0.5 * (jnp.cos(jnp.pi * dmin / CUT) + 1.0) * (d < CUT)
    return x / d, y / d, z / d, rbf, fcut


def _edge_geom_body(dv_ref, rbfcf_ref, rsh_ref):
    pid = pl.program_id(0)
    dv = dv_ref[...]
    ux, uy, uz, rbf, fcut = _geom_common(dv)
    gid = pid * BE + lax.broadcasted_iota(jnp.int32, (BE, 1), 0)
    mask = (gid < E).astype(jnp.float32)
    lanes = lax.broadcasted_iota(jnp.int32, (BE, RW), 1)
    fc = fcut * mask
    rbfcf = jnp.where(lanes < NB, rbf * fc, jnp.where(lanes == NB, fc, 0.0))
    rbfcf_ref[...] = rbfcf
    l16 = lax.broadcasted_iota(jnp.int32, (BE, SV), 1)
    r = jnp.where(l16 == 0, 1.0, 0.0)
    for i, t in enumerate([ux, uy, uz, ux * uy, uy * uz,
                           3.0 * uz * uz - 1.0, ux * uz, ux * ux - uy * uy]):
        r = jnp.where(l16 == i + 1, t, r)
    rsh_ref[...] = r


def _tc_edge_geom(dve):
    return _tc(
        _edge_geom_body, (EP // BE,),
        [_rows(BE, SV)],
        [_rows(BE, RW), _rows(BE, SV)],
        (jax.ShapeDtypeStruct((EP, RW), jnp.float32),
         jax.ShapeDtypeStruct((EP, SV), jnp.float32)),
    )(dve)


def _full_geom_body(dv_ref, wc_ref, gc_ref):
    pid = pl.program_id(0)
    dv = dv_ref[...]
    _, _, _, rbf, fcut = _geom_common(dv)
    gid = pid * BE + lax.broadcasted_iota(jnp.int32, (BE, 1), 0)
    mask = (gid < E).astype(jnp.float32)
    lanes = lax.broadcasted_iota(jnp.int32, (BE, RW), 1)
    frbf = jnp.where(lanes < NB, rbf * fcut * mask, 0.0)
    gc_ref[...] = jnp.dot(frbf, wc_ref[...],
                          preferred_element_type=jnp.float32)


def _tc_full_geom(dvf, wcomb):
    return _tc(
        _full_geom_body, (EP // BE,),
        [_rows(BE, SV), _full((RW, RW))],
        [_rows(BE, RW)],
        jax.ShapeDtypeStruct((EP, RW), jnp.float32),
    )(dvf, wcomb)


def _filt_body(rb_ref, rsh_ref, wr_ref, flts_ref, etab_ref):
    flt = jnp.dot(rb_ref[...], wr_ref[...], preferred_element_type=jnp.float32)
    flts_ref[...] = flt[:, 0:D]
    etab_ref[...] = jnp.concatenate(
        [flt[:, D:D + SV], flt[:, D + SV:W] * rsh_ref[...]], axis=1)


def _tc_filt(rbfcf, rsh, wrp):
    return _tc(
        _filt_body, (EP // BE,),
        [_rows(BE, RW), _rows(BE, SV), _full((RW, W))],
        [_rows(BE, D), _rows(BE, RW)],
        (jax.ShapeDtypeStruct((EP, D), jnp.float32),
         jax.ShapeDtypeStruct((EP, RW), jnp.float32)),
    )(rbfcf, rsh, wrp)


def _make_update_body(has_sph, emit_inv):
    def body(*refs):
        i = 0
        xs_ref = refs[i]; i += 1
        xv_ref = refs[i]; i += 1
        sagg_ref = refs[i]; i += 1
        vagg_ref = refs[i]; i += 1
        if has_sph == 2:
            nin_ref = refs[i]; i += 1
            xva_ref = refs[i]; i += 1
        up_ref = refs[i]; i += 1
        vp_ref = refs[i]; i += 1
        wu1_ref = refs[i]; i += 1
        wu2_ref = refs[i]; i += 1
        msc_ref = refs[i]; i += 1
        if emit_inv:
            w1n_ref = refs[i]; i += 1
            w2n_ref = refs[i]; i += 1
        xs_o = refs[i]; i += 1
        xv_o = refs[i]; i += 1
        if has_sph:
            nsph_o = refs[i]; i += 1
        if emit_inv:
            invs_o = refs[i]; i += 1
            vtab_o = refs[i]; i += 1
        if has_sph == 2:
            xvt_o = refs[i]; i += 1

        f32 = jnp.float32
        xs1 = xs_ref[...] + sagg_ref[0] + sagg_ref[1]
        mu = jnp.mean(xs1, axis=-1, keepdims=True)
        var = jnp.mean((xs1 - mu) ** 2, axis=-1, keepdims=True)
        xs1 = (xs1 - mu) / jnp.sqrt(var + 1e-5)
        xv1 = xv_ref[...] + vagg_ref[0] + vagg_ref[1]
        uv = jnp.dot(xv1, up_ref[...], preferred_element_type=f32)
        vv = jnp.dot(xv1, vp_ref[...], preferred_element_type=f32)
        vnorm = jnp.sqrt(jnp.sum(vv * vv, axis=-1, keepdims=True) + 1e-12)
        wu1b = msc_ref[0:1, 0:D]
        bu1 = msc_ref[1:2, 0:D]
        bu2 = msc_ref[2:3, :]
        t = _silu(jnp.dot(xs1, wu1_ref[...], preferred_element_type=f32)
                  + vnorm * wu1b + bu1)
        a = jnp.dot(t, wu2_ref[...], preferred_element_type=f32) + bu2
        dots = jnp.sum(uv * vv, axis=-1, keepdims=True)
        xs2 = xs1 + a[:, 0:D] + a[:, D:D + 1] * dots
        xv2 = xv1 + a[:, 144:160] * uv
        xs_o[...] = xs2
        xv_o[...] = xv2
        if has_sph:
            wn = msc_ref[3:4, 0:SV]
            ns = xv2 * wn
            if has_sph == 2:
                ns = ns + nin_ref[...]
            nsph_o[...] = ns
        if emit_inv:
            b1n = msc_ref[4:5, 0:D]
            b2n = msc_ref[5:6, :]
            tn = _silu(jnp.dot(xs2, w1n_ref[...], preferred_element_type=f32)
                       + b1n)
            invf = jnp.dot(tn, w2n_ref[...], preferred_element_type=f32) + b2n
            invs_o[...] = invf[:, 0:D]
            vtab_o[...] = jnp.concatenate(
                [invf[:, D:D + SV] * xv2, invf[:, D + SV:W],
                 jnp.zeros((BN, 96), f32)], axis=1)
        if has_sph == 2:
            xvt_o[...] = jnp.concatenate(
                [xva_ref[...], xv2, jnp.zeros((BN, 96), f32)], axis=1)
    return body


def _tc_update(has_sph, emit_inv, xs, xv, sagg, vagg, nin, up, vp, wu1, wu2,
               msc, w1n, w2n, xva=None):
    in_arrays = [xs, xv, sagg, vagg]
    in_specs = [_rows(BN, D), _rows(BN, SV),
                pl.BlockSpec((2, BN, D), lambda i: (0, i, 0)),
                pl.BlockSpec((2, BN, SV), lambda i: (0, i, 0))]
    if has_sph == 2:
        in_arrays += [nin, xva]
        in_specs += [_rows(BN, SV), _rows(BN, SV)]
    in_arrays += [up, vp, wu1, wu2, msc]
    in_specs += [_full((SV, SV)), _full((SV, SV)), _full((D, D)),
                 _full((D, W)), _full((8, W))]
    if emit_inv:
        in_arrays += [w1n, w2n]
        in_specs += [_full((D, D)), _full((D, W))]
    out_specs = [_rows(BN, D), _rows(BN, SV)]
    out_shapes = [jax.ShapeDtypeStruct((NP, D), jnp.float32),
                  jax.ShapeDtypeStruct((NP, SV), jnp.float32)]
    if has_sph:
        out_specs.append(_rows(BN, SV))
        out_shapes.append(jax.ShapeDtypeStruct((NP, SV), jnp.float32))
    if emit_inv:
        out_specs += [_rows(BN, D), _rows(BN, D)]
        out_shapes += [jax.ShapeDtypeStruct((NP, D), jnp.float32),
                       jax.ShapeDtypeStruct((NP, D), jnp.float32)]
    if has_sph == 2:
        out_specs.append(_rows(BN, D))
        out_shapes.append(jax.ShapeDtypeStruct((NP, D), jnp.float32))
    return _tc(_make_update_body(has_sph, emit_inv), (NP // BN,),
               in_specs, out_specs, tuple(out_shapes))(*in_arrays)


def _final_body(xs_ref, ns_ref, ea_ref, eb_ref, w1a_ref, w1b_ref, w2_ref,
                wep_ref, msc_ref, out_ref):
    f32 = jnp.float32
    b1 = msc_ref[0:1, :]
    b2 = msc_ref[1:2, 0:BLK]
    e9 = ea_ref[0] + ea_ref[1] + eb_ref[0] + eb_ref[1]
    nh = _silu(jnp.dot(xs_ref[...], w1a_ref[...], preferred_element_type=f32)
               + jnp.dot(ns_ref[...], w1b_ref[...], preferred_element_type=f32)
               + b1)
    out_ref[...] = (jnp.dot(nh, w2_ref[...], preferred_element_type=f32) + b2
                    + jnp.dot(e9, wep_ref[...], preferred_element_type=f32))


def _tc_final(xs, nsph, eagg, ebgg, w1a, w1b, w2, wep, msc):
    return _tc(
        _final_body, (NP // BN,),
        [_rows(BN, D), _rows(BN, SV),
         pl.BlockSpec((2, BN, SV), lambda i: (0, i, 0)),
         pl.BlockSpec((2, BN, SV), lambda i: (0, i, 0)),
         _full((D, HID)), _full((SV, HID)), _full((HID, BLK)),
         _full((SV, BLK)), _full((8, HID))],
        [_rows(BN, BLK)],
        jax.ShapeDtypeStruct((NP, BLK), jnp.float32),
    )(xs, nsph, eagg, ebgg, w1a, w1b, w2, wep, msc)


def _pad_cols(w, b):
    fi = w.shape[0]
    wp = jnp.zeros((fi, W), jnp.float32)
    wp = wp.at[:, 0:D].set(w[:, 0:D])
    wp = wp.at[:, D:D + SPH].set(w[:, D:D + SPH])
    wp = wp.at[:, 144:144 + SPH].set(w[:, D + SPH:D + 2 * SPH])
    bp = jnp.zeros((W,), jnp.float32)
    bp = bp.at[0:D].set(b[0:D])
    bp = bp.at[D:D + SPH].set(b[D:D + SPH])
    bp = bp.at[144:144 + SPH].set(b[D + SPH:D + 2 * SPH])
    return wp, bp


def kernel(at_no, pos, edge_index, edge_index_full, params):
    f32 = jnp.float32
    i32 = jnp.int32

    an_p = jnp.zeros((NP, 1), i32).at[:N, 0].set(at_no.astype(i32))
    pos_p = jnp.zeros((NR, SV), f32).at[:N, :3].set(pos)
    npad = EP - E
    spread = ((jnp.arange(npad, dtype=i32) * 37) % (N - 1)).astype(i32)
    def padidx(a):
        return jnp.concatenate([a.astype(i32), spread])
    se, de = padidx(edge_index[0]), padidx(edge_index[1])
    sf, df = padidx(edge_index_full[0]), padidx(edge_index_full[1])

    emb_p = jnp.zeros((D, D), f32).at[:100].set(params['embed'])
    msg, upd = [], []
    for i in range(3):
        pm = params['msg%d' % i]
        w2p, b2p = _pad_cols(pm['W2'], pm['b2'])
        wrp_, brp = _pad_cols(pm['Wr'], pm['br'])
        wrp = jnp.zeros((RW, W), f32).at[0:NB].set(wrp_).at[NB].set(brp)
        msg.append({'W1': pm['W1'], 'b1': pm['b1'], 'W2p': w2p, 'b2p': b2p,
                    'Wrp': wrp})
        pu = params['upd%d' % i]
        up = jnp.zeros((SV, SV), f32).at[:SPH, :SPH].set(pu['U'])
        vp = jnp.zeros((SV, SV), f32).at[:SPH, :SPH].set(pu['V'])
        wu2 = pu['Wu2']
        wu2p = jnp.zeros((D, W), f32)
        wu2p = wu2p.at[:, 0:D].set(wu2[:, 0:D])
        wu2p = wu2p.at[:, D:D + 1].set(wu2[:, D:D + 1])
        wu2p = wu2p.at[:, 144:144 + SPH].set(wu2[:, D + 1:D + 1 + SPH])
        bu2p = jnp.zeros((W,), f32)
        bu2p = bu2p.at[0:D].set(pu['bu2'][0:D])
        bu2p = bu2p.at[D:D + 1].set(pu['bu2'][D:D + 1])
        bu2p = bu2p.at[144:144 + SPH].set(pu['bu2'][D + 1:D + 1 + SPH])
        upd.append({'U': up, 'V': vp, 'Wu1a': pu['Wu1'][:D, :],
                    'wu1b': pu['Wu1'][D, :], 'bu1': pu['bu1'], 'bu2p': bu2p,
                    'Wu2p': wu2p})
    wn = [params['mat%d' % j]['wn'] for j in range(2)]
    wcomb = jnp.zeros((RW, RW), f32)
    wcomb = wcomb.at[0:NB, 0:SPH].set(params['mat0']['We'])
    wcomb = wcomb.at[0:NB, SV:SV + SPH].set(params['mat1']['We'])
    po = params['out']
    w1a = po['W1'][:D, :]
    w1b = jnp.zeros((SV, HID), f32).at[:SPH].set(po['W1'][D:D + SPH, :])
    wep = jnp.zeros((SV, BLK), f32).at[:SPH].set(po['We'])
    mscf = jnp.zeros((8, HID), f32).at[0].set(po['b1'])
    mscf = mscf.at[1, 0:BLK].set(po['b2'])

    def mk_misc(i):
        m = jnp.zeros((8, W), f32)
        m = m.at[0, 0:D].set(upd[i]['wu1b'])
        m = m.at[1, 0:D].set(upd[i]['bu1'])
        m = m.at[2].set(upd[i]['bu2p'])
        if i >= 1:
            m = m.at[3, 0:SPH].set(wn[i - 1])
        if i < 2:
            m = m.at[4, 0:D].set(msg[i + 1]['b1'])
            m = m.at[5].set(msg[i + 1]['b2p'])
        return m

    bv0 = jnp.zeros((8, W), f32).at[0, 0:D].set(msg[0]['b1'])
    bv0 = bv0.at[1].set(msg[0]['b2p'])

    dve, dvf = _sc_dvec(pos_p, se, de, sf, df)
    rbfcf, rsh = _tc_edge_geom(dve)
    gcomb = _tc_full_geom(dvf, wcomb)
    xs, invs, vtab = _tc_embed_inv(an_p, emb_p, msg[0]['W1'], msg[0]['W2p'],
                                   bv0)
    xv = jnp.zeros((NP, SV), f32)

    xva = None
    nsph = None
    xvtab = None
    for i in range(3):
        flts, etab = _tc_filt(rbfcf, rsh, msg[i]['Wrp'])
        sagg = _sc_msg(invs, flts, se, de)
        vagg = _sc_pair(vtab, etab, se, de)
        has_sph = 0 if i == 0 else (1 if i == 1 else 2)
        emit_inv = i < 2
        outs = _tc_update(has_sph, emit_inv, xs, xv, sagg, vagg, nsph,
                          upd[i]['U'], upd[i]['V'], upd[i]['Wu1a'],
                          upd[i]['Wu2p'], mk_misc(i),
                          msg[i + 1]['W1'] if emit_inv else None,
                          msg[i + 1]['W2p'] if emit_inv else None,
                          xva)
        outs = list(outs)
        xs, xv = outs[0], outs[1]
        k = 2
        if has_sph:
            nsph = outs[k]; k += 1
        if emit_inv:
            invs = outs[k]; k += 1
            vtab = outs[k]; k += 1
        if has_sph == 2:
            xvtab = outs[k]
        if i == 1:
            xva = xv

    ea1 = _sc_pair(xvtab, gcomb, sf, df)
    ea2 = _sc_pair(xvtab, gcomb, df, df)
    out = _tc_final(xs, nsph, ea1, ea2, w1a, w1b, po['W2'], wep, mscf)
    return out[:N]


# speedup vs baseline: 3.1069x; 1.0848x over previous
import functools

import jax
import jax.numpy as jnp
from jax import lax
from jax.experimental import pallas as pl
from jax.experimental.pallas import tpu as pltpu
from jax.experimental.pallas import tpu_sc as plsc

D = 128
NB = 20
CUT = 5.0
SPH = 9
SV = 16
W = 160
RW = 32
HID = 64
BLK = 32
N = 10000
NP = 10240
E = 160000
EP = 163840
CH = 64
NTILES = 32
EPT = EP // NTILES
NIT = EPT // CH
RPS = NP // 16
NR = 10112
RPS_S = NR // 16
BN = 1024
BE = 4096

@functools.cache
def _mesh():
    return plsc.VectorSubcoreMesh(core_axis_name="c", subcore_axis_name="s")


def _sig(x):
    return 1.0 / (1.0 + jnp.exp(-x))


def _silu(x):
    return x * _sig(x)


_SCHUNKS = [(k * CH, CH) for k in range(RPS_S // CH)] + [
    ((RPS_S // CH) * CH, RPS_S - (RPS_S // CH) * CH)]


def _sc_dvec_body(pos_h, se_h, de_h, sf_h, df_h, dve_h, dvf_h,
                  spos, ia0, ib0, ia1, ib1, pa0, pb0, pa1, pb1, dv0, dv1,
                  sem0, sem1, semi0, semi1, semo0, semo1):
    s = lax.axis_index("s")
    wid = s * 2 + lax.axis_index("c")
    slots = ((ia0, ib0, pa0, pb0, dv0, sem0, semi0, semo0),
             (ia1, ib1, pa1, pb1, dv1, sem1, semi1, semo1))
    for off, ln in _SCHUNKS:
        pltpu.sync_copy(pos_h.at[pl.ds(s * RPS_S + off, ln)],
                        pa0.at[pl.ds(0, ln)])
        pltpu.sync_copy(pa0.at[pl.ds(0, ln)],
                        spos.at[pl.ds(s * RPS_S + off, ln)])
    plsc.subcore_barrier()

    def do_list(src_h, dst_h, out_h):
        def issue_idx(slot, base):
            ia, ib, pa, pb, dv, sem, semi, semo = slot
            pltpu.async_copy(src_h.at[pl.ds(base, CH)], ia, semi)
            pltpu.async_copy(dst_h.at[pl.ds(base, CH)], ib, semi)

        def wait_idx(slot, base):
            ia, ib, pa, pb, dv, sem, semi, semo = slot
            pltpu.make_async_copy(src_h.at[pl.ds(base, CH)], ia, semi).wait()
            pltpu.make_async_copy(dst_h.at[pl.ds(base, CH)], ib, semi).wait()

        def issue_data(slot):
            ia, ib, pa, pb, dv, sem, semi, semo = slot
            pltpu.async_copy(spos.at[ia], pa, sem)
            pltpu.async_copy(spos.at[ib], pb, sem)

        def wait_data(slot):
            ia, ib, pa, pb, dv, sem, semi, semo = slot
            pltpu.make_async_copy(spos.at[ia], pa, sem).wait()
            pltpu.make_async_copy(spos.at[ib], pb, sem).wait()

        issue_idx(slots[0], wid * EPT)
        issue_idx(slots[1], wid * EPT + CH)
        wait_idx(slots[0], wid * EPT)
        issue_data(slots[0])
        wait_idx(slots[1], wid * EPT + CH)
        issue_data(slots[1])

        def it_body(k, cy):
            for b in range(2):
                g = 2 * k + b
                base = wid * EPT + g * CH
                slot = slots[b]
                ia, ib, pa, pb, dv, sem, semi, semo = slot
                wait_data(slot)

                @pl.when(k > 0)
                def _():
                    pltpu.make_async_copy(
                        dv, out_h.at[pl.ds(base, CH)], semo).wait()

                def row(r, carry):
                    for rr in range(2):
                        dv[2 * r + rr, :] = (pb[2 * r + rr, :]
                                             - pa[2 * r + rr, :])
                    return carry
                lax.fori_loop(0, CH // 2, row, 0)
                pltpu.async_copy(dv, out_h.at[pl.ds(base, CH)], semo)
                nxt = jnp.minimum(g + 2, NIT - 1)
                nbase = wid * EPT + nxt * CH
                issue_idx(slot, nbase)
                wait_idx(slot, nbase)
                issue_data(slot)
            return cy
        lax.fori_loop(0, NIT // 2, it_body, 0)
        last = wid * EPT + (NIT - 1) * CH
        for b in range(2):
            ia, ib, pa, pb, dv, sem, semi, semo = slots[b]
            wait_data(slots[b])
            pltpu.make_async_copy(dv, out_h.at[pl.ds(last, CH)], semo).wait()

    do_list(se_h, de_h, dve_h)
    do_list(sf_h, df_h, dvf_h)


def _sc_dvec(pos_p, se, de, sf, df):
    fn = pl.kernel(
        _sc_dvec_body,
        out_type=[jax.ShapeDtypeStruct((EP, SV), jnp.float32),
                  jax.ShapeDtypeStruct((EP, SV), jnp.float32)],
        mesh=_mesh(),
        scratch_types=[pltpu.VMEM_SHARED((NR, SV), jnp.float32)]
                      + [pltpu.VMEM((CH,), jnp.int32)] * 4
                      + [pltpu.VMEM((CH, SV), jnp.float32)] * 6
                      + [pltpu.SemaphoreType.DMA] * 6,
    )
    return fn(pos_p, se, de, sf, df)


def _sc_msg_body(inv_h, flts_h, src_h, dst_h, sagg_h,
                 accs, iv_s0, iv_d0, iv_s1, iv_d1, ivds, ginv0, gflt0,
                 ginv1, gflt1, ms, sem0, sem1, semi0, semi1):
    c = lax.axis_index("c")
    s = lax.axis_index("s")
    wid = s * 2 + c
    slots = ((iv_s0, iv_d0, ginv0, gflt0, sem0, semi0),
             (iv_s1, iv_d1, ginv1, gflt1, sem1, semi1))

    def zrow(r, carry):
        for j in range(8):
            ms[r, pl.ds(16 * j, 16)] = jnp.zeros((16,), jnp.float32)
        return carry
    lax.fori_loop(0, CH, zrow, 0)
    for off, ln in _SCHUNKS:
        pltpu.sync_copy(ms.at[pl.ds(0, ln)],
                        accs.at[pl.ds(s * RPS_S + off, ln)])
    plsc.subcore_barrier()

    def issue_idx(slot, base):
        ivs, ivd, gi, gf, sem, semi = slot
        pltpu.async_copy(src_h.at[pl.ds(base, CH)], ivs, semi)
        pltpu.async_copy(dst_h.at[pl.ds(base, CH)], ivd, semi)

    def wait_idx(slot, base):
        ivs, ivd, gi, gf, sem, semi = slot
        pltpu.make_async_copy(src_h.at[pl.ds(base, CH)], ivs, semi).wait()
        pltpu.make_async_copy(dst_h.at[pl.ds(base, CH)], ivd, semi).wait()

    def issue_data(slot, base):
        ivs, ivd, gi, gf, sem, semi = slot
        pltpu.async_copy(inv_h.at[ivs], gi, sem)
        pltpu.async_copy(flts_h.at[pl.ds(base, CH)], gf, sem)

    def wait_data(slot, base):
        ivs, ivd, gi, gf, sem, semi = slot
        pltpu.make_async_copy(inv_h.at[ivs], gi, sem).wait()
        pltpu.make_async_copy(flts_h.at[pl.ds(base, CH)], gf, sem).wait()

    issue_idx(slots[0], wid * EPT)
    issue_idx(slots[1], wid * EPT + CH)
    wait_idx(slots[0], wid * EPT)
    issue_data(slots[0], wid * EPT)
    wait_idx(slots[1], wid * EPT + CH)
    issue_data(slots[1], wid * EPT + CH)

    def it_body(k, _):
        for b in range(2):
            g = 2 * k + b
            base = wid * EPT + g * CH
            slot = slots[b]
            ivs, ivd, gi, gf, sem, semi = slot
            wait_data(slot, base)
            for q in range(CH // 16):
                ivds[pl.ds(16 * q, 16)] = ivd[pl.ds(16 * q, 16)]
            nxt = jnp.minimum(g + 2, NIT - 1)
            nbase = wid * EPT + nxt * CH
            issue_idx(slot, nbase)

            def row(r, carry):
                for rr in range(2):
                    for j in range(8):
                        ms[2 * r + rr, pl.ds(16 * j, 16)] = (
                            gi[2 * r + rr, pl.ds(16 * j, 16)]
                            * gf[2 * r + rr, pl.ds(16 * j, 16)])
                return carry
            lax.fori_loop(0, CH // 2, row, 0)
            pltpu.sync_copy(ms, accs.at[ivds], add=True)
            wait_idx(slot, nbase)
            issue_data(slot, nbase)
        return _
    lax.fori_loop(0, NIT // 2, it_body, 0)
    last = wid * EPT + (NIT - 1) * CH
    for b in range(2):
        wait_data(slots[b], last)
    plsc.subcore_barrier()
    for off, ln in _SCHUNKS:
        pltpu.sync_copy(accs.at[pl.ds(s * RPS_S + off, ln)],
                        ms.at[pl.ds(0, ln)])
        pltpu.sync_copy(ms.at[pl.ds(0, ln)],
                        sagg_h.at[c, pl.ds(s * RPS_S + off, ln)])


@functools.cache
def _sc_msg_fn():
    return pl.kernel(
        _sc_msg_body,
        out_type=[jax.ShapeDtypeStruct((2, NP, D), jnp.float32)],
        mesh=_mesh(),
        scratch_types=[pltpu.VMEM_SHARED((NR, D), jnp.float32)]
                      + [pltpu.VMEM((CH,), jnp.int32)] * 5
                      + [pltpu.VMEM((CH, D), jnp.float32)] * 5
                      + [pltpu.SemaphoreType.DMA] * 4,
    )


def _sc_msg(invs, flts, src, dst):
    return _sc_msg_fn()(invs, flts, src, dst)[0]


def _sc_pair_body(tab_h, gc_h, gi_h, si_h, agg_h,
                  acc, ig0, is0, ig1, is1, isd, gt0, gc0, gt1, gc1, ef,
                  sem0, sem1, semi0, semi1):
    c = lax.axis_index("c")
    s = lax.axis_index("s")
    wid = s * 2 + c
    slots = ((ig0, is0, gt0, gc0, sem0, semi0),
             (ig1, is1, gt1, gc1, sem1, semi1))

    def zrow(r, carry):
        ef[r, :] = jnp.zeros((16,), jnp.float32)
        return carry
    lax.fori_loop(0, CH, zrow, 0)
    for off, ln in _SCHUNKS:
        pltpu.sync_copy(ef.at[pl.ds(0, ln)],
                        acc.at[pl.ds(s * RPS_S + off, ln)])
    plsc.subcore_barrier()

    def issue_idx(slot, base):
        ivg, ivs, gt, gcv, sem, semi = slot
        pltpu.async_copy(gi_h.at[pl.ds(base, CH)], ivg, semi)
        pltpu.async_copy(si_h.at[pl.ds(base, CH)], ivs, semi)

    def wait_idx(slot, base):
        ivg, ivs, gt, gcv, sem, semi = slot
        pltpu.make_async_copy(gi_h.at[pl.ds(base, CH)], ivg, semi).wait()
        pltpu.make_async_copy(si_h.at[pl.ds(base, CH)], ivs, semi).wait()

    def issue_data(slot, base):
        ivg, ivs, gt, gcv, sem, semi = slot
        pltpu.async_copy(tab_h.at[ivg], gt, sem)
        pltpu.async_copy(gc_h.at[pl.ds(base, CH)], gcv, sem)

    def wait_data(slot, base):
        ivg, ivs, gt, gcv, sem, semi = slot
        pltpu.make_async_copy(tab_h.at[ivg], gt, sem).wait()
        pltpu.make_async_copy(gc_h.at[pl.ds(base, CH)], gcv, sem).wait()

    issue_idx(slots[0], wid * EPT)
    issue_idx(slots[1], wid * EPT + CH)
    wait_idx(slots[0], wid * EPT)
    issue_data(slots[0], wid * EPT)
    wait_idx(slots[1], wid * EPT + CH)
    issue_data(slots[1], wid * EPT + CH)

    def it_body(k, _):
        for b in range(2):
            g = 2 * k + b
            base = wid * EPT + g * CH
            slot = slots[b]
            ivg, ivs, gt, gcv, sem, semi = slot
            wait_data(slot, base)
            for q in range(CH // 16):
                isd[pl.ds(16 * q, 16)] = ivs[pl.ds(16 * q, 16)]
            nxt = jnp.minimum(g + 2, NIT - 1)
            nbase = wid * EPT + nxt * CH
            issue_idx(slot, nbase)

            def row(r, carry):
                for rr in range(2):
                    ef[2 * r + rr, :] = (
                        gcv[2 * r + rr, pl.ds(0, 16)]
                        * gt[2 * r + rr, pl.ds(0, 16)]
                        + gcv[2 * r + rr, pl.ds(16, 16)]
                        * gt[2 * r + rr, pl.ds(16, 16)])
                return carry
            lax.fori_loop(0, CH // 2, row, 0)
            pltpu.sync_copy(ef, acc.at[isd], add=True)
            wait_idx(slot, nbase)
            issue_data(slot, nbase)
        return _
    lax.fori_loop(0, NIT // 2, it_body, 0)
    last = wid * EPT + (NIT - 1) * CH
    for b in range(2):
        wait_data(slots[b], last)
    plsc.subcore_barrier()
    for off, ln in _SCHUNKS:
        pltpu.sync_copy(acc.at[pl.ds(s * RPS_S + off, ln)],
                        ef.at[pl.ds(0, ln)])
        pltpu.sync_copy(ef.at[pl.ds(0, ln)],
                        agg_h.at[c, pl.ds(s * RPS_S + off, ln)])


@functools.cache
def _sc_pair_fn():
    return pl.kernel(
        _sc_pair_body,
        out_type=[jax.ShapeDtypeStruct((2, NP, SV), jnp.float32)],
        mesh=_mesh(),
        scratch_types=[pltpu.VMEM_SHARED((NR, SV), jnp.float32)]
                      + [pltpu.VMEM((CH,), jnp.int32)] * 5
                      + [pltpu.VMEM((CH, D), jnp.float32),
                         pltpu.VMEM((CH, RW), jnp.float32),
                         pltpu.VMEM((CH, D), jnp.float32),
                         pltpu.VMEM((CH, RW), jnp.float32),
                         pltpu.VMEM((CH, SV), jnp.float32)]
                      + [pltpu.SemaphoreType.DMA] * 4,
    )


def _sc_pair(tab, gc, gidx, sidx):
    return _sc_pair_fn()(tab, gc, gidx, sidx)[0]


def _tc(body, grid, in_specs, out_specs, out_shapes):
    single = not isinstance(out_shapes, (list, tuple))
    if single:
        out_shapes = [out_shapes]
    fn = pl.pallas_call(
        body, grid=grid, in_specs=list(in_specs),
        out_specs=list(out_specs), out_shape=list(out_shapes))
    if single:
        return lambda *a: fn(*a)[0]
    return fn


def _full(shape):
    return pl.BlockSpec(shape, lambda i: (0,) * len(shape))


def _rows(bs, width):
    return pl.BlockSpec((bs, width), lambda i: (i, 0))


def _embed_inv_body(an_ref, emb_ref, w1_ref, w2_ref, bv_ref,
                    xs_ref, invs_ref, vtab_ref):
    an = an_ref[...]
    lanes = lax.broadcasted_iota(jnp.int32, (BN, D), 1)
    oh = (an == lanes).astype(jnp.float32)
    xs = jnp.dot(oh, emb_ref[...], preferred_element_type=jnp.float32)
    xs_ref[...] = xs
    b1 = bv_ref[0:1, 0:D]
    b2 = bv_ref[1:2, :]
    t = _silu(jnp.dot(xs, w1_ref[...], preferred_element_type=jnp.float32) + b1)
    inv = jnp.dot(t, w2_ref[...], preferred_element_type=jnp.float32) + b2
    invs_ref[...] = inv[:, 0:D]
    vtab_ref[...] = jnp.concatenate(
        [jnp.zeros((BN, SV), jnp.float32), inv[:, 144:160],
         jnp.zeros((BN, 96), jnp.float32)], axis=1)


def _tc_embed_inv(an_p, emb_p, w1, w2p, bv):
    return _tc(
        _embed_inv_body, (NP // BN,),
        [_rows(BN, 1), _full((D, D)), _full((D, D)), _full((D, W)),
         _full((8, W))],
        [_rows(BN, D), _rows(BN, D), _rows(BN, D)],
        (jax.ShapeDtypeStruct((NP, D), jnp.float32),
         jax.ShapeDtypeStruct((NP, D), jnp.float32),
         jax.ShapeDtypeStruct((NP, D), jnp.float32)),
    )(an_p, emb_p, w1, w2p, bv)


def _geom_common(dv):
    x = dv[:, 0:1]
    y = dv[:, 1:2]
    z = dv[:, 2:3]
    d = jnp.sqrt(x * x + y * y + z * z + 1e-12)
    nlane = (lax.broadcasted_iota(jnp.int32, (dv.shape[0], RW), 1)
             .astype(jnp.float32) + 1.0)
    rbf = jnp.sin(nlane * (jnp.pi / CUT) * d) / d
    dmin = jnp.minimum(d, CUT)
    fcut = 0.5 * (jnp.cos(jnp.pi * dmin / CUT) + 1.0) * (d < CUT)
    return x / d, y / d, z / d, rbf, fcut


def _edge_geom_body(dv_ref, rbfcf_ref, rsh_ref):
    pid = pl.program_id(0)
    dv = dv_ref[...]
    ux, uy, uz, rbf, fcut = _geom_common(dv)
    gid = pid * BE + lax.broadcasted_iota(jnp.int32, (BE, 1), 0)
    mask = (gid < E).astype(jnp.float32)
    lanes = lax.broadcasted_iota(jnp.int32, (BE, RW), 1)
    fc = fcut * mask
    rbfcf = jnp.where(lanes < NB, rbf * fc, jnp.where(lanes == NB, fc, 0.0))
    rbfcf_ref[...] = rbfcf
    l16 = lax.broadcasted_iota(jnp.int32, (BE, SV), 1)
    r = jnp.where(l16 == 0, 1.0, 0.0)
    for i, t in enumerate([ux, uy, uz, ux * uy, uy * uz,
                           3.0 * uz * uz - 1.0, ux * uz, ux * ux - uy * uy]):
        r = jnp.where(l16 == i + 1, t, r)
    rsh_ref[...] = r


def _tc_edge_geom(dve):
    return _tc(
        _edge_geom_body, (EP // BE,),
        [_rows(BE, SV)],
        [_rows(BE, RW), _rows(BE, SV)],
        (jax.ShapeDtypeStruct((EP, RW), jnp.float32),
         jax.ShapeDtypeStruct((EP, SV), jnp.float32)),
    )(dve)


def _full_geom_body(dv_ref, wc_ref, gc_ref):
    pid = pl.program_id(0)
    dv = dv_ref[...]
    _, _, _, rbf, fcut = _geom_common(dv)
    gid = pid * BE + lax.broadcasted_iota(jnp.int32, (BE, 1), 0)
    mask = (gid < E).astype(jnp.float32)
    lanes = lax.broadcasted_iota(jnp.int32, (BE, RW), 1)
    frbf = jnp.where(lanes < NB, rbf * fcut * mask, 0.0)
    gc_ref[...] = jnp.dot(frbf, wc_ref[...],
                          preferred_element_type=jnp.float32)


def _tc_full_geom(dvf, wcomb):
    return _tc(
        _full_geom_body, (EP // BE,),
        [_rows(BE, SV), _full((RW, RW))],
        [_rows(BE, RW)],
        jax.ShapeDtypeStruct((EP, RW), jnp.float32),
    )(dvf, wcomb)


def _filt_body(rb_ref, rsh_ref, wr_ref, flts_ref, etab_ref):
    flt = jnp.dot(rb_ref[...], wr_ref[...], preferred_element_type=jnp.float32)
    flts_ref[...] = flt[:, 0:D]
    etab_ref[...] = jnp.concatenate(
        [flt[:, D:D + SV], flt[:, D + SV:W] * rsh_ref[...]], axis=1)


def _tc_filt(rbfcf, rsh, wrp):
    return _tc(
        _filt_body, (EP // BE,),
        [_rows(BE, RW), _rows(BE, SV), _full((RW, W))],
        [_rows(BE, D), _rows(BE, RW)],
        (jax.ShapeDtypeStruct((EP, D), jnp.float32),
         jax.ShapeDtypeStruct((EP, RW), jnp.float32)),
    )(rbfcf, rsh, wrp)


def _make_update_body(has_sph, emit_inv):
    def body(*refs):
        i = 0
        xs_ref = refs[i]; i += 1
        xv_ref = refs[i]; i += 1
        sagg_ref = refs[i]; i += 1
        vagg_ref = refs[i]; i += 1
        if has_sph == 2:
            nin_ref = refs[i]; i += 1
            xva_ref = refs[i]; i += 1
        up_ref = refs[i]; i += 1
        vp_ref = refs[i]; i += 1
        wu1_ref = refs[i]; i += 1
        wu2_ref = refs[i]; i += 1
        msc_ref = refs[i]; i += 1
        if emit_inv:
            w1n_ref = refs[i]; i += 1
            w2n_ref = refs[i]; i += 1
        xs_o = refs[i]; i += 1
        xv_o = refs[i]; i += 1
        if has_sph:
            nsph_o = refs[i]; i += 1
        if emit_inv:
            invs_o = refs[i]; i += 1
            vtab_o = refs[i]; i += 1
        if has_sph == 2:
            xvt_o = refs[i]; i += 1

        f32 = jnp.float32
        xs1 = xs_ref[...] + sagg_ref[0] + sagg_ref[1]
        mu = jnp.mean(xs1, axis=-1, keepdims=True)
        var = jnp.mean((xs1 - mu) ** 2, axis=-1, keepdims=True)
        xs1 = (xs1 - mu) / jnp.sqrt(var + 1e-5)
        xv1 = xv_ref[...] + vagg_ref[0] + vagg_ref[1]
        uv = jnp.dot(xv1, up_ref[...], preferred_element_type=f32)
        vv = jnp.dot(xv1, vp_ref[...], preferred_element_type=f32)
        vnorm = jnp.sqrt(jnp.sum(vv * vv, axis=-1, keepdims=True) + 1e-12)
        wu1b = msc_ref[0:1, 0:D]
        bu1 = msc_ref[1:2, 0:D]
        bu2 = msc_ref[2:3, :]
        t = _silu(jnp.dot(xs1, wu1_ref[...], preferred_element_type=f32)
                  + vnorm * wu1b + bu1)
        a = jnp.dot(t, wu2_ref[...], preferred_element_type=f32) + bu2
        dots = jnp.sum(uv * vv, axis=-1, keepdims=True)
        xs2 = xs1 + a[:, 0:D] + a[:, D:D + 1] * dots
        xv2 = xv1 + a[:, 144:160] * uv
        xs_o[...] = xs2
        xv_o[...] = xv2
        if has_sph:
            wn = msc_ref[3:4, 0:SV]
            ns = xv2 * wn
            if has_sph == 2:
                ns = ns + nin_ref[...]
            nsph_o[...] = ns
        if emit_inv:
            b1n = msc_ref[4:5, 0:D]
            b2n = msc_ref[5:6, :]
            tn = _silu(jnp.dot(xs2, w1n_ref[...], preferred_element_type=f32)
                       + b1n)
            invf = jnp.dot(tn, w2n_ref[...], preferred_element_type=f32) + b2n
            invs_o[...] = invf[:, 0:D]
            vtab_o[...] = jnp.concatenate(
                [invf[:, D:D + SV] * xv2, invf[:, D + SV:W],
                 jnp.zeros((BN, 96), f32)], axis=1)
        if has_sph == 2:
            xvt_o[...] = jnp.concatenate(
                [xva_ref[...], xv2, jnp.zeros((BN, 96), f32)], axis=1)
    return body


def _tc_update(has_sph, emit_inv, xs, xv, sagg, vagg, nin, up, vp, wu1, wu2,
               msc, w1n, w2n, xva=None):
    in_arrays = [xs, xv, sagg, vagg]
    in_specs = [_rows(BN, D), _rows(BN, SV),
                pl.BlockSpec((2, BN, D), lambda i: (0, i, 0)),
                pl.BlockSpec((2, BN, SV), lambda i: (0, i, 0))]
    if has_sph == 2:
        in_arrays += [nin, xva]
        in_specs += [_rows(BN, SV), _rows(BN, SV)]
    in_arrays += [up, vp, wu1, wu2, msc]
    in_specs += [_full((SV, SV)), _full((SV, SV)), _full((D, D)),
                 _full((D, W)), _full((8, W))]
    if emit_inv:
        in_arrays += [w1n, w2n]
        in_specs += [_full((D, D)), _full((D, W))]
    out_specs = [_rows(BN, D), _rows(BN, SV)]
    out_shapes = [jax.ShapeDtypeStruct((NP, D), jnp.float32),
                  jax.ShapeDtypeStruct((NP, SV), jnp.float32)]
    if has_sph:
        out_specs.append(_rows(BN, SV))
        out_shapes.append(jax.ShapeDtypeStruct((NP, SV), jnp.float32))
    if emit_inv:
        out_specs += [_rows(BN, D), _rows(BN, D)]
        out_shapes += [jax.ShapeDtypeStruct((NP, D), jnp.float32),
                       jax.ShapeDtypeStruct((NP, D), jnp.float32)]
    if has_sph == 2:
        out_specs.append(_rows(BN, D))
        out_shapes.append(jax.ShapeDtypeStruct((NP, D), jnp.float32))
    return _tc(_make_update_body(has_sph, emit_inv), (NP // BN,),
               in_specs, out_specs, tuple(out_shapes))(*in_arrays)


def _final_body(xs_ref, ns_ref, ea_ref, eb_ref, w1a_ref, w1b_ref, w2_ref,
                wep_ref, msc_ref, out_ref):
    f32 = jnp.float32
    b1 = msc_ref[0:1, :]
    b2 = msc_ref[1:2, 0:BLK]
    e9 = ea_ref[0] + ea_ref[1] + eb_ref[0] + eb_ref[1]
    nh = _silu(jnp.dot(xs_ref[...], w1a_ref[...], preferred_element_type=f32)
               + jnp.dot(ns_ref[...], w1b_ref[...], preferred_element_type=f32)
               + b1)
    out_ref[...] = (jnp.dot(nh, w2_ref[...], preferred_element_type=f32) + b2
                    + jnp.dot(e9, wep_ref[...], preferred_element_type=f32))


def _tc_final(xs, nsph, eagg, ebgg, w1a, w1b, w2, wep, msc):
    return _tc(
        _final_body, (NP // BN,),
        [_rows(BN, D), _rows(BN, SV),
         pl.BlockSpec((2, BN, SV), lambda i: (0, i, 0)),
         pl.BlockSpec((2, BN, SV), lambda i: (0, i, 0)),
         _full((D, HID)), _full((SV, HID)), _full((HID, BLK)),
         _full((SV, BLK)), _full((8, HID))],
        [_rows(BN, BLK)],
        jax.ShapeDtypeStruct((NP, BLK), jnp.float32),
    )(xs, nsph, eagg, ebgg, w1a, w1b, w2, wep, msc)


def _pad_cols(w, b):
    fi = w.shape[0]
    wp = jnp.zeros((fi, W), jnp.float32)
    wp = wp.at[:, 0:D].set(w[:, 0:D])
    wp = wp.at[:, D:D + SPH].set(w[:, D:D + SPH])
    wp = wp.at[:, 144:144 + SPH].set(w[:, D + SPH:D + 2 * SPH])
    bp = jnp.zeros((W,), jnp.float32)
    bp = bp.at[0:D].set(b[0:D])
    bp = bp.at[D:D + SPH].set(b[D:D + SPH])
    bp = bp.at[144:144 + SPH].set(b[D + SPH:D + 2 * SPH])
    return wp, bp


def kernel(at_no, pos, edge_index, edge_index_full, params):
    f32 = jnp.float32
    i32 = jnp.int32

    an_p = jnp.zeros((NP, 1), i32).at[:N, 0].set(at_no.astype(i32))
    pos_p = jnp.zeros((NR, SV), f32).at[:N, :3].set(pos)
    npad = EP - E
    spread = ((jnp.arange(npad, dtype=i32) * 37) % (N - 1)).astype(i32)
    def padidx(a):
        return jnp.concatenate([a.astype(i32), spread])
    se, de = padidx(edge_index[0]), padidx(edge_index[1])
    sf, df = padidx(edge_index_full[0]), padidx(edge_index_full[1])

    emb_p = jnp.zeros((D, D), f32).at[:100].set(params['embed'])
    msg, upd = [], []
    for i in range(3):
        pm = params['msg%d' % i]
        w2p, b2p = _pad_cols(pm['W2'], pm['b2'])
        wrp_, brp = _pad_cols(pm['Wr'], pm['br'])
        wrp = jnp.zeros((RW, W), f32).at[0:NB].set(wrp_).at[NB].set(brp)
        msg.append({'W1': pm['W1'], 'b1': pm['b1'], 'W2p': w2p, 'b2p': b2p,
                    'Wrp': wrp})
        pu = params['upd%d' % i]
        up = jnp.zeros((SV, SV), f32).at[:SPH, :SPH].set(pu['U'])
        vp = jnp.zeros((SV, SV), f32).at[:SPH, :SPH].set(pu['V'])
        wu2 = pu['Wu2']
        wu2p = jnp.zeros((D, W), f32)
        wu2p = wu2p.at[:, 0:D].set(wu2[:, 0:D])
        wu2p = wu2p.at[:, D:D + 1].set(wu2[:, D:D + 1])
        wu2p = wu2p.at[:, 144:144 + SPH].set(wu2[:, D + 1:D + 1 + SPH])
        bu2p = jnp.zeros((W,), f32)
        bu2p = bu2p.at[0:D].set(pu['bu2'][0:D])
        bu2p = bu2p.at[D:D + 1].set(pu['bu2'][D:D + 1])
        bu2p = bu2p.at[144:144 + SPH].set(pu['bu2'][D + 1:D + 1 + SPH])
        upd.append({'U': up, 'V': vp, 'Wu1a': pu['Wu1'][:D, :],
                    'wu1b': pu['Wu1'][D, :], 'bu1': pu['bu1'], 'bu2p': bu2p,
                    'Wu2p': wu2p})
    wn = [params['mat%d' % j]['wn'] for j in range(2)]
    wcomb = jnp.zeros((RW, RW), f32)
    wcomb = wcomb.at[0:NB, 0:SPH].set(params['mat0']['We'])
    wcomb = wcomb.at[0:NB, SV:SV + SPH].set(params['mat1']['We'])
    po = params['out']
    w1a = po['W1'][:D, :]
    w1b = jnp.zeros((SV, HID), f32).at[:SPH].set(po['W1'][D:D + SPH, :])
    wep = jnp.zeros((SV, BLK), f32).at[:SPH].set(po['We'])
    mscf = jnp.zeros((8, HID), f32).at[0].set(po['b1'])
    mscf = mscf.at[1, 0:BLK].set(po['b2'])

    def mk_misc(i):
        m = jnp.zeros((8, W), f32)
        m = m.at[0, 0:D].set(upd[i]['wu1b'])
        m = m.at[1, 0:D].set(upd[i]['bu1'])
        m = m.at[2].set(upd[i]['bu2p'])
        if i >= 1:
            m = m.at[3, 0:SPH].set(wn[i - 1])
        if i < 2:
            m = m.at[4, 0:D].set(msg[i + 1]['b1'])
            m = m.at[5].set(msg[i + 1]['b2p'])
        return m

    bv0 = jnp.zeros((8, W), f32).at[0, 0:D].set(msg[0]['b1'])
    bv0 = bv0.at[1].set(msg[0]['b2p'])

    dve, dvf = _sc_dvec(pos_p, se, de, sf, df)
    rbfcf, rsh = _tc_edge_geom(dve)
    gcomb = _tc_full_geom(dvf, wcomb)
    xs, invs, vtab = _tc_embed_inv(an_p, emb_p, msg[0]['W1'], msg[0]['W2p'],
                                   bv0)
    xv = jnp.zeros((NP, SV), f32)

    xva = None
    nsph = None
    xvtab = None
    for i in range(3):
        flts, etab = _tc_filt(rbfcf, rsh, msg[i]['Wrp'])
        sagg = _sc_msg(invs, flts, se, de)
        vagg = _sc_pair(vtab, etab, se, de)
        has_sph = 0 if i == 0 else (1 if i == 1 else 2)
        emit_inv = i < 2
        outs = _tc_update(has_sph, emit_inv, xs, xv, sagg, vagg, nsph,
                          upd[i]['U'], upd[i]['V'], upd[i]['Wu1a'],
                          upd[i]['Wu2p'], mk_misc(i),
                          msg[i + 1]['W1'] if emit_inv else None,
                          msg[i + 1]['W2p'] if emit_inv else None,
                          xva)
        outs = list(outs)
        xs, xv = outs[0], outs[1]
        k = 2
        if has_sph:
            nsph = outs[k]; k += 1
        if emit_inv:
            invs = outs[k]; k += 1
            vtab = outs[k]; k += 1
        if has_sph == 2:
            xvtab = outs[k]
        if i == 1:
            xva = xv

    ea1 = _sc_pair(xvtab, gcomb, sf, df)
    ea2 = _sc_pair(xvtab, gcomb, df, df)
    out = _tc_final(xs, nsph, ea1, ea2, w1a, w1b, po['W2'], wep, mscf)
    return out[:N]
